# baseline (speedup 1.0000x reference)
import jax
import jax.numpy as jnp
from jax import lax
from jax.experimental import pallas as pl
from jax.experimental.pallas import tpu as pltpu
from jax.experimental.pallas import tpu_sc as plsc

N = 25600
E = 409600
IN = 128
OUT = 128
NC = 40
MAXN = 10

NCORE = 2
NSUB = 16
TILES = NCORE * NSUB
EPT = E // TILES
NPT = N // TILES
CHUNK = 128
NCH = EPT // CHUNK
GN = 8
GROWS = GN * MAXN
NGCH = NPT // GN
NBSZ = N * MAXN + TILES * CHUNK

_MESH = plsc.VectorSubcoreMesh(core_axis_name="c", subcore_axis_name="s",
                               num_cores=NCORE, num_subcores=NSUB)
_SC_PARAMS = pltpu.CompilerParams(needs_layout_passes=False)


def _wid():
    return lax.axis_index("s") * NCORE + lax.axis_index("c")


def _tc0_body(h_ref, fcw_ref, bmat_ref, cmat_ref, z_ref, aux_ref):
    h = h_ref[...]
    z = jnp.dot(h, fcw_ref[...], preferred_element_type=jnp.float32)
    z_ref[...] = z
    aux = (jnp.dot(z, bmat_ref[...], preferred_element_type=jnp.float32)
           + jnp.dot(h, cmat_ref[...], preferred_element_type=jnp.float32))
    aux_ref[...] = aux


def _tc0(h, fc_w, bmat, cmat):
    blk = 1600
    grid = (N // blk,)
    return pl.pallas_call(
        _tc0_body,
        grid=grid,
        in_specs=[
            pl.BlockSpec((blk, IN), lambda i: (i, 0)),
            pl.BlockSpec((IN, OUT), lambda i: (0, 0)),
            pl.BlockSpec((OUT, 8), lambda i: (0, 0)),
            pl.BlockSpec((IN, 8), lambda i: (0, 0)),
        ],
        out_specs=[
            pl.BlockSpec((blk, OUT), lambda i: (i, 0)),
            pl.BlockSpec((blk, 8), lambda i: (i, 0)),
        ],
        out_shape=[
            jax.ShapeDtypeStruct((N, OUT), jnp.float32),
            jax.ShapeDtypeStruct((N, 8), jnp.float32),
        ],
    )(h, fc_w, bmat, cmat)


def _sc1_body(src_hbm, dst_hbm, histd_out, hists_out, src_v, dst_v, hd_v, hs_v):
    w = _wid()
    pltpu.sync_copy(src_hbm.at[pl.ds(w * EPT, EPT)], src_v)
    pltpu.sync_copy(dst_hbm.at[pl.ds(w * EPT, EPT)], dst_v)

    zeros16 = jnp.zeros((16,), jnp.int32)

    def zbody(i, _):
        hd_v[pl.ds(i * 16, 16)] = zeros16
        hs_v[pl.ds(i * 16, 16)] = zeros16
        return 0

    lax.fori_loop(0, N // 16, zbody, 0)

    def body(v, _):
        dv = dst_v[pl.ds(v * 16, 16)]
        cnt, last = plsc.scan_count(dv)
        base = plsc.load_gather(hd_v, [dv])
        plsc.store_scatter(hd_v, [dv], base + cnt, mask=last)
        sv = src_v[pl.ds(v * 16, 16)]
        cnt2, last2 = plsc.scan_count(sv)
        base2 = plsc.load_gather(hs_v, [sv])
        plsc.store_scatter(hs_v, [sv], base2 + cnt2, mask=last2)
        return 0

    lax.fori_loop(0, EPT // 16, body, 0)
    pltpu.sync_copy(hd_v, histd_out.at[pl.ds(w * N, N)])
    pltpu.sync_copy(hs_v, hists_out.at[pl.ds(w * N, N)])


def _sc1(src, dst):
    f = pl.kernel(
        _sc1_body,
        out_type=(jax.ShapeDtypeStruct((TILES * N,), jnp.int32),
                  jax.ShapeDtypeStruct((TILES * N,), jnp.int32)),
        mesh=_MESH,
        compiler_params=_SC_PARAMS,
        scratch_types=[
            pltpu.VMEM((EPT,), jnp.int32),
            pltpu.VMEM((EPT,), jnp.int32),
            pltpu.VMEM((N,), jnp.int32),
            pltpu.VMEM((N,), jnp.int32),
        ],
    )
    return f(src, dst)


def _tc1_body(histd_ref, hists_ref, hw_ref, offs_ref, di_ref, df_ref):
    hd = histd_ref[...]
    run = jnp.zeros_like(hd[0:1, :])
    rows = []
    for t in range(TILES):
        rows.append(run)
        run = run + hd[t:t + 1, :]
    offs_ref[...] = jnp.concatenate(rows, axis=0)
    deg_in = run
    deg_out = jnp.sum(hists_ref[...], axis=0, keepdims=True)
    nsrc = lax.rsqrt(jnp.maximum(deg_out, 1).astype(jnp.float32))
    nd = lax.rsqrt(jnp.maximum(deg_in, 1).astype(jnp.float32))
    msgval = hw_ref[...] * nsrc
    min_deg = jnp.minimum(jnp.maximum(deg_in, 1), MAXN)
    di_ref[...] = jnp.concatenate([deg_in, min_deg], axis=0)
    df_ref[...] = jnp.concatenate([msgval, nd], axis=0)


def _tc1(histd, hists, hw_row):
    blk = 3200
    grid = (N // blk,)
    return pl.pallas_call(
        _tc1_body,
        grid=grid,
        in_specs=[
            pl.BlockSpec((TILES, blk), lambda i: (0, i)),
            pl.BlockSpec((TILES, blk), lambda i: (0, i)),
            pl.BlockSpec((1, blk), lambda i: (0, i)),
        ],
        out_specs=[
            pl.BlockSpec((TILES, blk), lambda i: (0, i)),
            pl.BlockSpec((2, blk), lambda i: (0, i)),
            pl.BlockSpec((2, blk), lambda i: (0, i)),
        ],
        out_shape=[
            jax.ShapeDtypeStruct((TILES, N), jnp.int32),
            jax.ShapeDtypeStruct((2, N), jnp.int32),
            jax.ShapeDtypeStruct((2, N), jnp.float32),
        ],
    )(histd, hists, hw_row)


ZC = NBSZ // NSUB
ZB = 2032


def _sc2_body(src_hbm, dst_hbm, offs_hbm, msg_hbm, nb_out, aggp_out,
              src_v, dst_v, cnt_v, msg_v, agg_v, widx_v, widxB_v, zero_v,
              nb_sh, semA, semB):
    s = lax.axis_index("s")
    core = lax.axis_index("c")
    w = s * NCORE + core
    pltpu.sync_copy(src_hbm.at[pl.ds(w * EPT, EPT)], src_v)
    pltpu.sync_copy(dst_hbm.at[pl.ds(w * EPT, EPT)], dst_v)
    pltpu.sync_copy(offs_hbm.at[pl.ds(w * N, N)], cnt_v)
    pltpu.sync_copy(msg_hbm, msg_v)

    zeros16f = jnp.zeros((16,), jnp.float32)
    zeros16 = jnp.zeros((16,), jnp.int32)

    def zbody(i, _):
        agg_v[pl.ds(i * 16, 16)] = zeros16f
        return 0

    lax.fori_loop(0, N // 16, zbody, 0)

    def zbody2(i, _):
        zero_v[pl.ds(i * 16, 16)] = zeros16
        return 0

    lax.fori_loop(0, ZB // 16, zbody2, 0)
    for i in range(ZC // ZB):
        pltpu.sync_copy(zero_v, nb_sh.at[pl.ds(s * ZC + i * ZB, ZB)])
    plsc.subcore_barrier()

    iota16 = lax.iota(jnp.int32, 16)

    def chunk_work(p, cr, widx_b, sem):
        @pl.when(p > 0)
        def _():
            pltpu.make_async_copy(src_v.at[pl.ds((cr - 2) * CHUNK, CHUNK)],
                                  nb_sh.at[widx_b], sem).wait()

        for k in range(CHUNK // 16):
            dv = dst_v[pl.ds(cr * CHUNK + k * 16, 16)]
            cntv, lastv = plsc.scan_count(dv)
            rank_i = cntv - 1
            base = plsc.load_gather(cnt_v, [dv])
            plsc.store_scatter(cnt_v, [dv], base + cntv, mask=lastv)
            rank = base + rank_i
            ok = rank < MAXN
            dump = N * MAXN + w * CHUNK + k * 16 + iota16
            widx = jnp.where(ok, dv * MAXN + rank, dump)
            widx_b[pl.ds(k * 16, 16)] = widx
            sv = src_v[pl.ds(cr * CHUNK + k * 16, 16)]
            mv = plsc.load_gather(msg_v, [sv])
            nround = jnp.max(cntv)

            def rbody(r, _):
                plsc.addupdate_scatter(agg_v, [dv], mv, mask=(rank_i == r))
                return 0

            lax.fori_loop(0, nround, rbody, 0)
        pltpu.async_copy(src_v.at[pl.ds(cr * CHUNK, CHUNK)],
                         nb_sh.at[widx_b], sem, add=True)

    def body(p, _):
        chunk_work(p, p * 2, widx_v, semA)
        chunk_work(p, p * 2 + 1, widxB_v, semB)
        return 0

    lax.fori_loop(0, NCH // 2, body, 0)
    pltpu.make_async_copy(src_v.at[pl.ds((NCH - 2) * CHUNK, CHUNK)],
                          nb_sh.at[widx_v], semA).wait()
    pltpu.make_async_copy(src_v.at[pl.ds((NCH - 1) * CHUNK, CHUNK)],
                          nb_sh.at[widxB_v], semB).wait()
    pltpu.sync_copy(agg_v, aggp_out.at[pl.ds(w * N, N)])
    plsc.subcore_barrier()

    @pl.when(s == 0)
    def _():
        pltpu.sync_copy(nb_sh, nb_out.at[pl.ds(core * NBSZ, NBSZ)])


def _sc2(src, dst, offs_flat, msgval):
    f = pl.kernel(
        _sc2_body,
        out_type=(jax.ShapeDtypeStruct((NCORE * NBSZ,), jnp.int32),
                  jax.ShapeDtypeStruct((TILES * N,), jnp.float32)),
        mesh=_MESH,
        compiler_params=_SC_PARAMS,
        scratch_types=[
            pltpu.VMEM((EPT,), jnp.int32),
            pltpu.VMEM((EPT,), jnp.int32),
            pltpu.VMEM((N,), jnp.int32),
            pltpu.VMEM((N,), jnp.float32),
            pltpu.VMEM((N,), jnp.float32),
            pltpu.VMEM((CHUNK,), jnp.int32),
            pltpu.VMEM((CHUNK,), jnp.int32),
            pltpu.VMEM((ZB,), jnp.int32),
            pltpu.VMEM_SHARED((NBSZ,), jnp.int32),
            pltpu.SemaphoreType.DMA,
            pltpu.SemaphoreType.DMA,
        ],
    )
    return f(src, dst, offs_flat, msgval)


def _sc3a_body(za1_hbm, za2_hbm, zw0_hbm, zw1_hbm, deg_hbm, mind_hbm,
               nb_hbm, cc_hbm, rowt_out, idx_out, cw_out,
               za1_v, zw0_v, zw1_v, za2_v, deg_v, mind_v, nbl_v, nblb_v,
               row_v, idx_v, cw_v, cc_v):
    w = _wid()
    base = w * NPT
    pltpu.sync_copy(za1_hbm, za1_v)
    pltpu.sync_copy(zw0_hbm, zw0_v)
    pltpu.sync_copy(zw1_hbm, zw1_v)
    pltpu.sync_copy(za2_hbm.at[pl.ds(base, NPT)], za2_v)
    pltpu.sync_copy(deg_hbm.at[pl.ds(base, NPT)], deg_v)
    pltpu.sync_copy(mind_hbm.at[pl.ds(base, NPT)], mind_v)
    pltpu.sync_copy(nb_hbm.at[pl.ds(base * MAXN, NPT * MAXN)], nbl_v)
    pltpu.sync_copy(nb_hbm.at[pl.ds(NBSZ + base * MAXN, NPT * MAXN)], nblb_v)
    pltpu.sync_copy(cc_hbm, cc_v)

    def mbody(i, _):
        nbl_v[pl.ds(i * 16, 16)] = (nbl_v[pl.ds(i * 16, 16)]
                                    + nblb_v[pl.ds(i * 16, 16)])
        return 0

    lax.fori_loop(0, NPT * MAXN // 16, mbody, 0)

    iota16 = lax.iota(jnp.int32, 16)
    cc_all = cc_v[...]

    def body(g, _):
        loc = g * 16 + iota16
        nabs = base + loc
        degv = deg_v[pl.ds(g * 16, 16)]
        mdv = mind_v[pl.ds(g * 16, 16)]
        za2v = za2_v[pl.ds(g * 16, 16)]
        isolated = degv <= 0

        nbs = []
        es = []
        for j in range(MAXN):
            jj = lax.rem(jnp.full((16,), j, jnp.int32), mdv)
            nbj = plsc.load_gather(nbl_v, [loc * MAXN + jj])
            nbj = jnp.where(isolated, nabs, nbj)
            nbs.append(nbj)
            t = plsc.load_gather(za1_v, [nbj]) + za2v
            es.append(jnp.where(t >= 0, t, 0.01 * t))
        mx = es[0]
        for j in range(1, MAXN):
            mx = jnp.maximum(mx, es[j])
        exs = [jnp.exp(es[j] - mx) for j in range(MAXN)]
        s = exs[0]
        for j in range(1, MAXN):
            s = s + exs[j]
        inv = 1.0 / s
        alphas = [exs[j] * inv for j in range(MAXN)]

        w0g = [plsc.load_gather(zw0_v, [nbs[j]]) for j in range(MAXN - 1)]
        w1g = [None] + [plsc.load_gather(zw1_v, [nbs[j]]) for j in range(1, MAXN)]
        for i in range(MAXN - 1):
            row_v[pl.ds(i * NPT + g * 16, 16)] = (alphas[i] * w0g[i]
                                                  + alphas[i + 1] * w1g[i + 1])
        for j in range(MAXN):
            sidx = loc * MAXN + j
            plsc.store_scatter(idx_v, [sidx], nbs[j])
            plsc.store_scatter(cw_v, [sidx], alphas[j] * cc_all[j])
        return 0

    lax.fori_loop(0, NPT // 16, body, 0)
    for i in range(MAXN - 1):
        pltpu.sync_copy(row_v.at[pl.ds(i * NPT, NPT)],
                        rowt_out.at[pl.ds(i * N + base, NPT)])
    pltpu.sync_copy(idx_v, idx_out.at[pl.ds(base * MAXN, NPT * MAXN)])
    pltpu.sync_copy(cw_v, cw_out.at[pl.ds(base * MAXN, NPT * MAXN)])


def _sc3a(za1, za2, zw0, zw1, deg, mind, nb, cc):
    f = pl.kernel(
        _sc3a_body,
        out_type=(jax.ShapeDtypeStruct(((MAXN - 1) * N,), jnp.float32),
                  jax.ShapeDtypeStruct((N * MAXN,), jnp.int32),
                  jax.ShapeDtypeStruct((N * MAXN,), jnp.float32)),
        mesh=_MESH,
        compiler_params=_SC_PARAMS,
        scratch_types=[
            pltpu.VMEM((N,), jnp.float32),
            pltpu.VMEM((N,), jnp.float32),
            pltpu.VMEM((N,), jnp.float32),
            pltpu.VMEM((NPT,), jnp.float32),
            pltpu.VMEM((NPT,), jnp.int32),
            pltpu.VMEM((NPT,), jnp.int32),
            pltpu.VMEM((NPT * MAXN,), jnp.int32),
            pltpu.VMEM((NPT * MAXN,), jnp.int32),
            pltpu.VMEM(((MAXN - 1) * NPT,), jnp.float32),
            pltpu.VMEM((NPT * MAXN,), jnp.int32),
            pltpu.VMEM((NPT * MAXN,), jnp.float32),
            pltpu.VMEM((16,), jnp.float32),
        ],
    )
    return f(za1, za2, zw0, zw1, deg, mind, nb, cc)


def _sc3b_body(z_hbm, idx_hbm, cw_hbm, col_out,
               idx_v, cw_v, zb0_v, zb1_v, zb2_v, zb3_v,
               colstA_v, colstB_v, colstC_v, colstD_v,
               sem0, sem1, sem2, sem3, semo):
    colsts = [colstA_v, colstB_v, colstC_v, colstD_v]
    w = _wid()
    eb = w * NPT * MAXN
    pltpu.sync_copy(idx_hbm.at[pl.ds(eb, NPT * MAXN)], idx_v)
    pltpu.sync_copy(cw_hbm.at[pl.ds(eb, NPT * MAXN)], cw_v)

    def compute(c, zbuf_v, colst_v):
        wv = [cw_v[pl.ds(c * GROWS + t * 16, 16)] for t in range(GROWS // 16)]
        for nl in range(GN):
            accs = [jnp.zeros((16,), jnp.float32) for _ in range(OUT // 16)]
            for j in range(MAXN):
                r = nl * MAXN + j
                wgt = wv[r // 16][r % 16]
                for q in range(OUT // 16):
                    accs[q] = accs[q] + zbuf_v[r, pl.ds(q * 16, 16)] * wgt
            for q in range(OUT // 16):
                colst_v[nl, pl.ds(q * 16, 16)] = accs[q]

    def fire(c, zbuf_v, sem):
        pltpu.async_copy(z_hbm.at[idx_v.at[pl.ds(c * GROWS, GROWS)]],
                         zbuf_v, sem)

    def drain(c, zbuf_v, sem):
        pltpu.make_async_copy(z_hbm.at[idx_v.at[pl.ds(c * GROWS, GROWS)]],
                              zbuf_v, sem).wait()

    def out_slice(c):
        return col_out.at[pl.ds(w * NPT + c * GN, GN)]

    zbs = [zb0_v, zb1_v, zb2_v, zb3_v]
    gsems = [sem0, sem1, sem2, sem3]
    for c in range(3):
        fire(c, zbs[c], gsems[c])

    def quarter(p, c, zbuf_v, gsem, colst_v):
        drain(c, zbuf_v, gsem)

        @pl.when(p > 0)
        def _():
            pltpu.make_async_copy(colst_v, out_slice(c - 4), semo).wait()

        compute(c, zbuf_v, colst_v)
        pltpu.async_copy(colst_v, out_slice(c), semo)

    def body(p, _):
        c0 = p * 4
        for l in range(4):
            c = c0 + l
            ln = (l + 3) % 4

            @pl.when(c + 3 < NGCH)
            def _():
                fire(c + 3, zbs[ln], gsems[ln])

            quarter(p, c, zbs[l], gsems[l], colsts[l])
        return 0

    lax.fori_loop(0, NGCH // 4, body, 0)
    for l in range(4):
        pltpu.make_async_copy(colsts[l], out_slice(NGCH - 4 + l), semo).wait()


def _sc3b(z, idx_flat, cw_flat):
    f = pl.kernel(
        _sc3b_body,
        out_type=jax.ShapeDtypeStruct((N, OUT), jnp.float32),
        mesh=_MESH,
        compiler_params=_SC_PARAMS,
        scratch_types=[
            pltpu.VMEM((NPT * MAXN,), jnp.int32),
            pltpu.VMEM((NPT * MAXN,), jnp.float32),
            pltpu.VMEM((GROWS, OUT), jnp.float32),
            pltpu.VMEM((GROWS, OUT), jnp.float32),
            pltpu.VMEM((GROWS, OUT), jnp.float32),
            pltpu.VMEM((GROWS, OUT), jnp.float32),
            pltpu.VMEM((GN, OUT), jnp.float32),
            pltpu.VMEM((GN, OUT), jnp.float32),
            pltpu.VMEM((GN, OUT), jnp.float32),
            pltpu.VMEM((GN, OUT), jnp.float32),
            pltpu.SemaphoreType.DMA,
            pltpu.SemaphoreType.DMA,
            pltpu.SemaphoreType.DMA,
            pltpu.SemaphoreType.DMA,
            pltpu.SemaphoreType.DMA,
        ],
    )
    return f(z, idx_flat, cw_flat)


def _tc3a_body(rowt_ref, col_ref, crb_ref, ccb_ref, part_ref):
    r = rowt_ref[...] + crb_ref[0, 0]
    c = col_ref[...] + ccb_ref[0, 0]
    sr = jnp.sum(r)
    ssr = jnp.sum(r * r)
    sc = jnp.sum(c)
    ssc = jnp.sum(c * c)
    row0 = jnp.concatenate(
        [x.reshape(1, 1) for x in (sr, ssr, sc, ssc)]
        + [jnp.zeros((1, 124), jnp.float32)], axis=1)
    part_ref[...] = jnp.concatenate(
        [row0, jnp.zeros((7, 128), jnp.float32)], axis=0)


def _tc3a(rowt, col, crb, ccb):
    blk = 3200
    grid = (N // blk,)
    return pl.pallas_call(
        _tc3a_body,
        grid=grid,
        in_specs=[
            pl.BlockSpec((MAXN - 1, blk), lambda i: (0, i)),
            pl.BlockSpec((blk, OUT), lambda i: (i, 0)),
            pl.BlockSpec((1, 1), lambda i: (0, 0)),
            pl.BlockSpec((1, 1), lambda i: (0, 0)),
        ],
        out_specs=pl.BlockSpec((8, 128), lambda i: (i, 0)),
        out_shape=jax.ShapeDtypeStruct((N // blk * 8, 128), jnp.float32),
    )(rowt, col, crb, ccb)


def _tc3b_body(rowt_ref, col_ref, h_ref, aggp_ref, df_ref, part_ref,
               crb_ref, ccb_ref, bn1g_ref, bn1b_ref, bn2g_ref, bn2b_ref,
               gcb_ref, lw9_ref, lw128_ref, clsw_ref, clsb_ref,
               out_ref, acc_ref):
    i = pl.program_id(0)
    nblk = pl.num_programs(0)
    blk = col_ref.shape[0]

    part = part_ref[...]
    sr = jnp.sum(part[:, 0])
    ssr = jnp.sum(part[:, 1])
    sc = jnp.sum(part[:, 2])
    ssc = jnp.sum(part[:, 3])
    nr = float(N * (MAXN - 1))
    ncl = float(N * OUT)
    mu1 = sr / nr
    var1 = ssr / nr - mu1 * mu1
    mu2 = sc / ncl
    var2 = ssc / ncl - mu2 * mu2
    inv1 = bn1g_ref[0, 0] * lax.rsqrt(var1 + 1e-5)
    inv2 = bn2g_ref[0, 0] * lax.rsqrt(var2 + 1e-5)

    r = rowt_ref[...] + crb_ref[0, 0]
    r1 = jnp.maximum((r - mu1) * inv1 + bn1b_ref[0, 0], 0.0)
    c = col_ref[...] + ccb_ref[0, 0]
    c1 = jnp.maximum((c - mu2) * inv2 + bn2b_ref[0, 0], 0.0)

    gat9 = lax.dot_general(r1, lw9_ref[...], (((0,), (0,)), ((), ())),
                           preferred_element_type=jnp.float32)
    gatc = jnp.dot(c1, lw128_ref[...], preferred_element_type=jnp.float32)
    feats = jnp.maximum(gat9 + gatc + h_ref[...], 0.0)

    agg = jnp.sum(aggp_ref[...], axis=0, keepdims=True)
    gc = agg * df_ref[1:2, :] + gcb_ref[0, 0]
    ng = blk // IN
    gcr = jnp.concatenate([gc[:, k * IN:(k + 1) * IN] for k in range(ng)],
                          axis=0)
    gcr = gcr - jnp.max(gcr, axis=1, keepdims=True)
    egc = jnp.exp(gcr)
    gw = egc / jnp.sum(egc, axis=1, keepdims=True)
    contrib = jnp.zeros((1, OUT), jnp.float32)
    for k in range(ng):
        contrib = contrib + jnp.dot(gw[k:k + 1, :],
                                    feats[k * IN:(k + 1) * IN, :],
                                    preferred_element_type=jnp.float32)

    newacc = jnp.where(i == 0, contrib, acc_ref[...] + contrib)
    acc_ref[...] = newacc

    @pl.when(i == nblk - 1)
    def _():
        hg = newacc / float(N)
        out_ref[...] = lax.dot_general(
            hg, clsw_ref[...], (((1,), (1,)), ((), ())),
            preferred_element_type=jnp.float32) + clsb_ref[...]


def _tc3b(rowt, col, h, aggp, dmisc_f, part, crb, ccb, bn1g, bn1b, bn2g,
          bn2b, gcb, lw9, lw128, clsw, clsb):
    blk = 3200
    grid = (N // blk,)
    return pl.pallas_call(
        _tc3b_body,
        grid=grid,
        in_specs=[
            pl.BlockSpec((MAXN - 1, blk), lambda i: (0, i)),
            pl.BlockSpec((blk, OUT), lambda i: (i, 0)),
            pl.BlockSpec((blk, IN), lambda i: (i, 0)),
            pl.BlockSpec((TILES, blk), lambda i: (0, i)),
            pl.BlockSpec((2, blk), lambda i: (0, i)),
            pl.BlockSpec((N // blk * 8, 128), lambda i: (0, 0)),
            pl.BlockSpec((1, 1), lambda i: (0, 0)),
            pl.BlockSpec((1, 1), lambda i: (0, 0)),
            pl.BlockSpec((1, 1), lambda i: (0, 0)),
            pl.BlockSpec((1, 1), lambda i: (0, 0)),
            pl.BlockSpec((1, 1), lambda i: (0, 0)),
            pl.BlockSpec((1, 1), lambda i: (0, 0)),
            pl.BlockSpec((1, 1), lambda i: (0, 0)),
            pl.BlockSpec((MAXN - 1, OUT), lambda i: (0, 0)),
            pl.BlockSpec((OUT, OUT), lambda i: (0, 0)),
            pl.BlockSpec((NC, OUT), lambda i: (0, 0)),
            pl.BlockSpec((1, NC), lambda i: (0, 0)),
        ],
        out_specs=pl.BlockSpec((1, NC), lambda i: (0, 0)),
        out_shape=jax.ShapeDtypeStruct((1, NC), jnp.float32),
        scratch_shapes=[pltpu.VMEM((1, OUT), jnp.float32)],
    )(rowt, col, h, aggp, dmisc_f, part, crb, ccb, bn1g, bn1b, bn2g, bn2b,
      gcb, lw9, lw128, clsw, clsb)


def kernel(h, edge_index, fc_w, attn_w, convrow_w, convrow_b, bn1_g, bn1_b,
           convcol_w, convcol_b, bn2_g, bn2_b, gc_w, gc_b, localw, cls_w,
           cls_b):
    src = edge_index[0]
    dst = edge_index[1]

    a1 = attn_w[:OUT, 0]
    a2 = attn_w[OUT:, 0]
    w0 = convrow_w[0, 0, 0, :]
    w1 = convrow_w[0, 0, 1, :]
    zero = jnp.zeros((OUT,), jnp.float32)
    bmat = jnp.stack([a1, a2, w0, w1, zero, zero, zero, zero], axis=1)
    cmat = jnp.stack([zero, zero, zero, zero, gc_w[:, 0], zero, zero, zero],
                     axis=1)

    z, aux = _tc0(h, fc_w, bmat, cmat)
    za1 = aux[:, 0]
    za2 = aux[:, 1]
    zw0 = aux[:, 2]
    zw1 = aux[:, 3]
    hw_row = aux[:, 4].reshape(1, N)

    histd_flat, hists_flat = _sc1(src, dst)
    offs, dmisc_i, dmisc_f = _tc1(histd_flat.reshape(TILES, N),
                                  hists_flat.reshape(TILES, N), hw_row)

    nb, aggp_flat = _sc2(src, dst, offs.reshape(-1), dmisc_f[0])

    cc = jnp.pad(convcol_w[0, 0, :, 0], (0, 16 - MAXN))
    rowt_flat, idx_flat, cw_flat = _sc3a(za1, za2, zw0, zw1,
                                         dmisc_i[0], dmisc_i[1], nb, cc)

    col = _sc3b(z, idx_flat, cw_flat)

    rowt = rowt_flat.reshape(MAXN - 1, N)
    aggp = aggp_flat.reshape(TILES, N)

    crb = convrow_b.reshape(1, 1)
    ccb = convcol_b.reshape(1, 1)
    part = _tc3a(rowt, col, crb, ccb)

    out = _tc3b(rowt, col, h, aggp, dmisc_f, part, crb, ccb,
                bn1_g.reshape(1, 1), bn1_b.reshape(1, 1),
                bn2_g.reshape(1, 1), bn2_b.reshape(1, 1),
                gc_b.reshape(1, 1), localw[:MAXN - 1], localw[MAXN - 1:],
                cls_w, cls_b.reshape(1, NC))
    return out


# speedup vs baseline: 1.0091x; 1.0091x over previous
import jax
import jax.numpy as jnp
from jax import lax
from jax.experimental import pallas as pl
from jax.experimental.pallas import tpu as pltpu
from jax.experimental.pallas import tpu_sc as plsc

N = 25600
E = 409600
IN = 128
OUT = 128
NC = 40
MAXN = 10

NCORE = 2
NSUB = 16
TILES = NCORE * NSUB
EPT = E // TILES
NPT = N // TILES
CHUNK = 128
NCH = EPT // CHUNK
GN = 8
GROWS = GN * MAXN
NGCH = NPT // GN
NBSZ = N * MAXN + TILES * CHUNK

_MESH = plsc.VectorSubcoreMesh(core_axis_name="c", subcore_axis_name="s",
                               num_cores=NCORE, num_subcores=NSUB)
_SC_PARAMS = pltpu.CompilerParams(needs_layout_passes=False)


def _wid():
    return lax.axis_index("s") * NCORE + lax.axis_index("c")


def _tc0_body(h_ref, fcw_ref, bmat_ref, cmat_ref, z_ref, aux_ref):
    h = h_ref[...]
    z = jnp.dot(h, fcw_ref[...], preferred_element_type=jnp.float32)
    z_ref[...] = z
    aux = (jnp.dot(z, bmat_ref[...], preferred_element_type=jnp.float32)
           + jnp.dot(h, cmat_ref[...], preferred_element_type=jnp.float32))
    aux_ref[...] = aux


def _tc0(h, fc_w, bmat, cmat):
    blk = 1600
    grid = (N // blk,)
    return pl.pallas_call(
        _tc0_body,
        grid=grid,
        in_specs=[
            pl.BlockSpec((blk, IN), lambda i: (i, 0)),
            pl.BlockSpec((IN, OUT), lambda i: (0, 0)),
            pl.BlockSpec((OUT, 8), lambda i: (0, 0)),
            pl.BlockSpec((IN, 8), lambda i: (0, 0)),
        ],
        out_specs=[
            pl.BlockSpec((blk, OUT), lambda i: (i, 0)),
            pl.BlockSpec((blk, 8), lambda i: (i, 0)),
        ],
        out_shape=[
            jax.ShapeDtypeStruct((N, OUT), jnp.float32),
            jax.ShapeDtypeStruct((N, 8), jnp.float32),
        ],
    )(h, fc_w, bmat, cmat)


def _sc1_body(src_hbm, dst_hbm, histd_out, hists_out, src_v, dst_v, hd_v, hs_v):
    w = _wid()
    pltpu.sync_copy(src_hbm.at[pl.ds(w * EPT, EPT)], src_v)
    pltpu.sync_copy(dst_hbm.at[pl.ds(w * EPT, EPT)], dst_v)

    zeros16 = jnp.zeros((16,), jnp.int32)

    def zbody(i, _):
        hd_v[pl.ds(i * 16, 16)] = zeros16
        hs_v[pl.ds(i * 16, 16)] = zeros16
        return 0

    lax.fori_loop(0, N // 16, zbody, 0)

    def body(v, _):
        dv = dst_v[pl.ds(v * 16, 16)]
        cnt, last = plsc.scan_count(dv)
        base = plsc.load_gather(hd_v, [dv])
        plsc.store_scatter(hd_v, [dv], base + cnt, mask=last)
        sv = src_v[pl.ds(v * 16, 16)]
        cnt2, last2 = plsc.scan_count(sv)
        base2 = plsc.load_gather(hs_v, [sv])
        plsc.store_scatter(hs_v, [sv], base2 + cnt2, mask=last2)
        return 0

    lax.fori_loop(0, EPT // 16, body, 0)
    pltpu.sync_copy(hd_v, histd_out.at[pl.ds(w * N, N)])
    pltpu.sync_copy(hs_v, hists_out.at[pl.ds(w * N, N)])


def _sc1(src, dst):
    f = pl.kernel(
        _sc1_body,
        out_type=(jax.ShapeDtypeStruct((TILES * N,), jnp.int32),
                  jax.ShapeDtypeStruct((TILES * N,), jnp.int32)),
        mesh=_MESH,
        compiler_params=_SC_PARAMS,
        scratch_types=[
            pltpu.VMEM((EPT,), jnp.int32),
            pltpu.VMEM((EPT,), jnp.int32),
            pltpu.VMEM((N,), jnp.int32),
            pltpu.VMEM((N,), jnp.int32),
        ],
    )
    return f(src, dst)


def _tc1_body(histd_ref, hists_ref, hw_ref, offs_ref, di_ref, df_ref):
    hd = histd_ref[...]
    run = jnp.zeros_like(hd[0:1, :])
    rows = []
    for t in range(TILES):
        rows.append(run)
        run = run + hd[t:t + 1, :]
    offs_ref[...] = jnp.concatenate(rows, axis=0)
    deg_in = run
    deg_out = jnp.sum(hists_ref[...], axis=0, keepdims=True)
    nsrc = lax.rsqrt(jnp.maximum(deg_out, 1).astype(jnp.float32))
    nd = lax.rsqrt(jnp.maximum(deg_in, 1).astype(jnp.float32))
    msgval = hw_ref[...] * nsrc
    min_deg = jnp.minimum(jnp.maximum(deg_in, 1), MAXN)
    di_ref[...] = jnp.concatenate([deg_in, min_deg], axis=0)
    df_ref[...] = jnp.concatenate([msgval, nd], axis=0)


def _tc1(histd, hists, hw_row):
    blk = 3200
    grid = (N // blk,)
    return pl.pallas_call(
        _tc1_body,
        grid=grid,
        in_specs=[
            pl.BlockSpec((TILES, blk), lambda i: (0, i)),
            pl.BlockSpec((TILES, blk), lambda i: (0, i)),
            pl.BlockSpec((1, blk), lambda i: (0, i)),
        ],
        out_specs=[
            pl.BlockSpec((TILES, blk), lambda i: (0, i)),
            pl.BlockSpec((2, blk), lambda i: (0, i)),
            pl.BlockSpec((2, blk), lambda i: (0, i)),
        ],
        out_shape=[
            jax.ShapeDtypeStruct((TILES, N), jnp.int32),
            jax.ShapeDtypeStruct((2, N), jnp.int32),
            jax.ShapeDtypeStruct((2, N), jnp.float32),
        ],
    )(histd, hists, hw_row)


ZC = NBSZ // NSUB
ZB = 2032


def _sc2_body(src_hbm, dst_hbm, offs_hbm, msg_hbm, nb_out, aggp_out,
              src_v, dst_v, cnt_v, msg_v, agg_v, widx_v, widxB_v, zero_v,
              nb_sh, semA, semB):
    s = lax.axis_index("s")
    core = lax.axis_index("c")
    w = s * NCORE + core
    pltpu.sync_copy(src_hbm.at[pl.ds(w * EPT, EPT)], src_v)
    pltpu.sync_copy(dst_hbm.at[pl.ds(w * EPT, EPT)], dst_v)
    pltpu.sync_copy(offs_hbm.at[pl.ds(w * N, N)], cnt_v)
    pltpu.sync_copy(msg_hbm, msg_v)

    zeros16f = jnp.zeros((16,), jnp.float32)
    zeros16 = jnp.zeros((16,), jnp.int32)

    def zbody(i, _):
        agg_v[pl.ds(i * 16, 16)] = zeros16f
        return 0

    lax.fori_loop(0, N // 16, zbody, 0)

    def zbody2(i, _):
        zero_v[pl.ds(i * 16, 16)] = zeros16
        return 0

    lax.fori_loop(0, ZB // 16, zbody2, 0)
    for i in range(ZC // ZB):
        pltpu.sync_copy(zero_v, nb_sh.at[pl.ds(s * ZC + i * ZB, ZB)])
    plsc.subcore_barrier()

    iota16 = lax.iota(jnp.int32, 16)

    def chunk_work(p, cr, widx_b, sem):
        @pl.when(p > 0)
        def _():
            pltpu.make_async_copy(src_v.at[pl.ds((cr - 2) * CHUNK, CHUNK)],
                                  nb_sh.at[widx_b], sem).wait()

        for k in range(CHUNK // 16):
            dv = dst_v[pl.ds(cr * CHUNK + k * 16, 16)]
            cntv, lastv = plsc.scan_count(dv)
            rank_i = cntv - 1
            base = plsc.load_gather(cnt_v, [dv])
            plsc.store_scatter(cnt_v, [dv], base + cntv, mask=lastv)
            rank = base + rank_i
            ok = rank < MAXN
            dump = N * MAXN + w * CHUNK + k * 16 + iota16
            widx = jnp.where(ok, dv * MAXN + rank, dump)
            widx_b[pl.ds(k * 16, 16)] = widx
            sv = src_v[pl.ds(cr * CHUNK + k * 16, 16)]
            mv = plsc.load_gather(msg_v, [sv])
            plsc.addupdate_scatter(agg_v, [dv], mv, mask=(rank_i == 0))
            ndup = plsc.all_reduce_population_count(rank_i > 0)

            @pl.when(ndup[0] > 0)
            def _():
                def rbody(r, _):
                    plsc.addupdate_scatter(agg_v, [dv], mv,
                                           mask=(rank_i == r))
                    return 0

                lax.fori_loop(1, jnp.max(cntv), rbody, 0)
        pltpu.async_copy(src_v.at[pl.ds(cr * CHUNK, CHUNK)],
                         nb_sh.at[widx_b], sem, add=True)

    def body(p, _):
        chunk_work(p, p * 2, widx_v, semA)
        chunk_work(p, p * 2 + 1, widxB_v, semB)
        return 0

    lax.fori_loop(0, NCH // 2, body, 0)
    pltpu.make_async_copy(src_v.at[pl.ds((NCH - 2) * CHUNK, CHUNK)],
                          nb_sh.at[widx_v], semA).wait()
    pltpu.make_async_copy(src_v.at[pl.ds((NCH - 1) * CHUNK, CHUNK)],
                          nb_sh.at[widxB_v], semB).wait()
    pltpu.sync_copy(agg_v, aggp_out.at[pl.ds(w * N, N)])
    plsc.subcore_barrier()

    @pl.when(s == 0)
    def _():
        pltpu.sync_copy(nb_sh, nb_out.at[pl.ds(core * NBSZ, NBSZ)])


def _sc2(src, dst, offs_flat, msgval):
    f = pl.kernel(
        _sc2_body,
        out_type=(jax.ShapeDtypeStruct((NCORE * NBSZ,), jnp.int32),
                  jax.ShapeDtypeStruct((TILES * N,), jnp.float32)),
        mesh=_MESH,
        compiler_params=_SC_PARAMS,
        scratch_types=[
            pltpu.VMEM((EPT,), jnp.int32),
            pltpu.VMEM((EPT,), jnp.int32),
            pltpu.VMEM((N,), jnp.int32),
            pltpu.VMEM((N,), jnp.float32),
            pltpu.VMEM((N,), jnp.float32),
            pltpu.VMEM((CHUNK,), jnp.int32),
            pltpu.VMEM((CHUNK,), jnp.int32),
            pltpu.VMEM((ZB,), jnp.int32),
            pltpu.VMEM_SHARED((NBSZ,), jnp.int32),
            pltpu.SemaphoreType.DMA,
            pltpu.SemaphoreType.DMA,
        ],
    )
    return f(src, dst, offs_flat, msgval)


def _sc3a_body(za1_hbm, za2_hbm, zw0_hbm, zw1_hbm, deg_hbm, mind_hbm,
               nb_hbm, cc_hbm, rowt_out, idx_out, cw_out,
               za1_v, zw0_v, zw1_v, za2_v, deg_v, mind_v, nbl_v, nblb_v,
               row_v, idx_v, cw_v, cc_v):
    w = _wid()
    base = w * NPT
    pltpu.sync_copy(za1_hbm, za1_v)
    pltpu.sync_copy(zw0_hbm, zw0_v)
    pltpu.sync_copy(zw1_hbm, zw1_v)
    pltpu.sync_copy(za2_hbm.at[pl.ds(base, NPT)], za2_v)
    pltpu.sync_copy(deg_hbm.at[pl.ds(base, NPT)], deg_v)
    pltpu.sync_copy(mind_hbm.at[pl.ds(base, NPT)], mind_v)
    pltpu.sync_copy(nb_hbm.at[pl.ds(base * MAXN, NPT * MAXN)], nbl_v)
    pltpu.sync_copy(nb_hbm.at[pl.ds(NBSZ + base * MAXN, NPT * MAXN)], nblb_v)
    pltpu.sync_copy(cc_hbm, cc_v)

    def mbody(i, _):
        nbl_v[pl.ds(i * 16, 16)] = (nbl_v[pl.ds(i * 16, 16)]
                                    + nblb_v[pl.ds(i * 16, 16)])
        return 0

    lax.fori_loop(0, NPT * MAXN // 16, mbody, 0)

    iota16 = lax.iota(jnp.int32, 16)
    cc_all = cc_v[...]

    def body(g, _):
        loc = g * 16 + iota16
        nabs = base + loc
        degv = deg_v[pl.ds(g * 16, 16)]
        mdv = mind_v[pl.ds(g * 16, 16)]
        za2v = za2_v[pl.ds(g * 16, 16)]
        isolated = degv <= 0

        nbs = []
        es = []
        for j in range(MAXN):
            jj = lax.rem(jnp.full((16,), j, jnp.int32), mdv)
            nbj = plsc.load_gather(nbl_v, [loc * MAXN + jj])
            nbj = jnp.where(isolated, nabs, nbj)
            nbs.append(nbj)
            t = plsc.load_gather(za1_v, [nbj]) + za2v
            es.append(jnp.where(t >= 0, t, 0.01 * t))
        mx = es[0]
        for j in range(1, MAXN):
            mx = jnp.maximum(mx, es[j])
        exs = [jnp.exp(es[j] - mx) for j in range(MAXN)]
        s = exs[0]
        for j in range(1, MAXN):
            s = s + exs[j]
        inv = 1.0 / s
        alphas = [exs[j] * inv for j in range(MAXN)]

        w0g = [plsc.load_gather(zw0_v, [nbs[j]]) for j in range(MAXN - 1)]
        w1g = [None] + [plsc.load_gather(zw1_v, [nbs[j]]) for j in range(1, MAXN)]
        for i in range(MAXN - 1):
            row_v[pl.ds(i * NPT + g * 16, 16)] = (alphas[i] * w0g[i]
                                                  + alphas[i + 1] * w1g[i + 1])
        for j in range(MAXN):
            sidx = loc * MAXN + j
            plsc.store_scatter(idx_v, [sidx], nbs[j])
            plsc.store_scatter(cw_v, [sidx], alphas[j] * cc_all[j])
        return 0

    lax.fori_loop(0, NPT // 16, body, 0)
    for i in range(MAXN - 1):
        pltpu.sync_copy(row_v.at[pl.ds(i * NPT, NPT)],
                        rowt_out.at[pl.ds(i * N + base, NPT)])
    pltpu.sync_copy(idx_v, idx_out.at[pl.ds(base * MAXN, NPT * MAXN)])
    pltpu.sync_copy(cw_v, cw_out.at[pl.ds(base * MAXN, NPT * MAXN)])


def _sc3a(za1, za2, zw0, zw1, deg, mind, nb, cc):
    f = pl.kernel(
        _sc3a_body,
        out_type=(jax.ShapeDtypeStruct(((MAXN - 1) * N,), jnp.float32),
                  jax.ShapeDtypeStruct((N * MAXN,), jnp.int32),
                  jax.ShapeDtypeStruct((N * MAXN,), jnp.float32)),
        mesh=_MESH,
        compiler_params=_SC_PARAMS,
        scratch_types=[
            pltpu.VMEM((N,), jnp.float32),
            pltpu.VMEM((N,), jnp.float32),
            pltpu.VMEM((N,), jnp.float32),
            pltpu.VMEM((NPT,), jnp.float32),
            pltpu.VMEM((NPT,), jnp.int32),
            pltpu.VMEM((NPT,), jnp.int32),
            pltpu.VMEM((NPT * MAXN,), jnp.int32),
            pltpu.VMEM((NPT * MAXN,), jnp.int32),
            pltpu.VMEM(((MAXN - 1) * NPT,), jnp.float32),
            pltpu.VMEM((NPT * MAXN,), jnp.int32),
            pltpu.VMEM((NPT * MAXN,), jnp.float32),
            pltpu.VMEM((16,), jnp.float32),
        ],
    )
    return f(za1, za2, zw0, zw1, deg, mind, nb, cc)


def _sc3b_body(z_hbm, idx_hbm, cw_hbm, col_out,
               idx_v, cw_v, zb0_v, zb1_v, colstA_v, colstB_v,
               sem0, sem1, semo):
    w = _wid()
    eb = w * NPT * MAXN
    pltpu.sync_copy(idx_hbm.at[pl.ds(eb, NPT * MAXN)], idx_v)
    pltpu.sync_copy(cw_hbm.at[pl.ds(eb, NPT * MAXN)], cw_v)

    def compute(c, zbuf_v, colst_v):
        wv = [cw_v[pl.ds(c * GROWS + t * 16, 16)] for t in range(GROWS // 16)]
        for nl in range(GN):
            accs = [jnp.zeros((16,), jnp.float32) for _ in range(OUT // 16)]
            for j in range(MAXN):
                r = nl * MAXN + j
                wgt = wv[r // 16][r % 16]
                for q in range(OUT // 16):
                    accs[q] = accs[q] + zbuf_v[r, pl.ds(q * 16, 16)] * wgt
            for q in range(OUT // 16):
                colst_v[nl, pl.ds(q * 16, 16)] = accs[q]

    def fire(c, zbuf_v, sem):
        pltpu.async_copy(z_hbm.at[idx_v.at[pl.ds(c * GROWS, GROWS)]],
                         zbuf_v, sem)

    def drain(c, zbuf_v, sem):
        pltpu.make_async_copy(z_hbm.at[idx_v.at[pl.ds(c * GROWS, GROWS)]],
                              zbuf_v, sem).wait()

    def out_slice(c):
        return col_out.at[pl.ds(w * NPT + c * GN, GN)]

    fire(0, zb0_v, sem0)
    fire(1, zb1_v, sem1)

    def half(p, c, zbuf_v, gsem, colst_v):
        drain(c, zbuf_v, gsem)

        @pl.when(p > 0)
        def _():
            pltpu.make_async_copy(colst_v, out_slice(c - 2), semo).wait()

        compute(c, zbuf_v, colst_v)
        pltpu.async_copy(colst_v, out_slice(c), semo)

        @pl.when(c + 2 < NGCH)
        def _():
            fire(c + 2, zbuf_v, gsem)

    def body(p, _):
        c0 = p * 2
        half(p, c0, zb0_v, sem0, colstA_v)
        half(p, c0 + 1, zb1_v, sem1, colstB_v)
        return 0

    lax.fori_loop(0, NGCH // 2, body, 0)
    pltpu.make_async_copy(colstA_v, out_slice(NGCH - 2), semo).wait()
    pltpu.make_async_copy(colstB_v, out_slice(NGCH - 1), semo).wait()


def _sc3b(z, idx_flat, cw_flat):
    f = pl.kernel(
        _sc3b_body,
        out_type=jax.ShapeDtypeStruct((N, OUT), jnp.float32),
        mesh=_MESH,
        compiler_params=_SC_PARAMS,
        scratch_types=[
            pltpu.VMEM((NPT * MAXN,), jnp.int32),
            pltpu.VMEM((NPT * MAXN,), jnp.float32),
            pltpu.VMEM((GROWS, OUT), jnp.float32),
            pltpu.VMEM((GROWS, OUT), jnp.float32),
            pltpu.VMEM((GN, OUT), jnp.float32),
            pltpu.VMEM((GN, OUT), jnp.float32),
            pltpu.SemaphoreType.DMA,
            pltpu.SemaphoreType.DMA,
            pltpu.SemaphoreType.DMA,
        ],
    )
    return f(z, idx_flat, cw_flat)


def _tc3_body(rowt_ref, col_ref, h_ref, aggp_ref, df_ref,
              crb_ref, ccb_ref, bn1g_ref, bn1b_ref, bn2g_ref, bn2b_ref,
              gcb_ref, lw9_ref, lw128_ref, clsw_ref, clsb_ref,
              out_ref, acc_ref, st_ref):
    i = pl.program_id(0)
    nblk = pl.num_programs(0) // 2
    blk = col_ref.shape[0]

    r = rowt_ref[...] + crb_ref[0, 0]
    c = col_ref[...] + ccb_ref[0, 0]

    @pl.when(i < nblk)
    def _():
        stats = jnp.concatenate(
            [x.reshape(1, 1) for x in
             (jnp.sum(r), jnp.sum(r * r), jnp.sum(c), jnp.sum(c * c))]
            + [jnp.zeros((1, 124), jnp.float32)], axis=1)
        st_ref[...] = jnp.where(i == 0, stats, st_ref[...] + stats)

    @pl.when(i >= nblk)
    def _():
        st = st_ref[...]
        nr = float(N * (MAXN - 1))
        ncl = float(N * OUT)
        mu1 = st[0, 0] / nr
        var1 = st[0, 1] / nr - mu1 * mu1
        mu2 = st[0, 2] / ncl
        var2 = st[0, 3] / ncl - mu2 * mu2
        inv1 = bn1g_ref[0, 0] * lax.rsqrt(var1 + 1e-5)
        inv2 = bn2g_ref[0, 0] * lax.rsqrt(var2 + 1e-5)

        r1 = jnp.maximum((r - mu1) * inv1 + bn1b_ref[0, 0], 0.0)
        c1 = jnp.maximum((c - mu2) * inv2 + bn2b_ref[0, 0], 0.0)

        gat9 = lax.dot_general(r1, lw9_ref[...], (((0,), (0,)), ((), ())),
                               preferred_element_type=jnp.float32)
        gatc = jnp.dot(c1, lw128_ref[...], preferred_element_type=jnp.float32)
        feats = jnp.maximum(gat9 + gatc + h_ref[...], 0.0)

        agg = jnp.sum(aggp_ref[...], axis=0, keepdims=True)
        gc = agg * df_ref[1:2, :] + gcb_ref[0, 0]
        ng = blk // IN
        gcr = jnp.concatenate([gc[:, k * IN:(k + 1) * IN] for k in range(ng)],
                              axis=0)
        gcr = gcr - jnp.max(gcr, axis=1, keepdims=True)
        egc = jnp.exp(gcr)
        gw = egc / jnp.sum(egc, axis=1, keepdims=True)
        contrib = jnp.zeros((1, OUT), jnp.float32)
        for k in range(ng):
            contrib = contrib + jnp.dot(gw[k:k + 1, :],
                                        feats[k * IN:(k + 1) * IN, :],
                                        preferred_element_type=jnp.float32)

        newacc = jnp.where(i == nblk, contrib, acc_ref[...] + contrib)
        acc_ref[...] = newacc

        @pl.when(i == 2 * nblk - 1)
        def _():
            hg = newacc / float(N)
            out_ref[...] = lax.dot_general(
                hg, clsw_ref[...], (((1,), (1,)), ((), ())),
                preferred_element_type=jnp.float32) + clsb_ref[...]


def _tc3(rowt, col, h, aggp, dmisc_f, crb, ccb, bn1g, bn1b, bn2g,
         bn2b, gcb, lw9, lw128, clsw, clsb):
    blk = 3200
    nblk = N // blk
    grid = (2 * nblk,)
    bi = lambda i: (i % nblk, 0)
    bj = lambda i: (0, i % nblk)
    z = lambda i: (0, 0)
    return pl.pallas_call(
        _tc3_body,
        grid=grid,
        in_specs=[
            pl.BlockSpec((MAXN - 1, blk), bj),
            pl.BlockSpec((blk, OUT), bi),
            pl.BlockSpec((blk, IN), bi),
            pl.BlockSpec((TILES, blk), bj),
            pl.BlockSpec((2, blk), bj),
            pl.BlockSpec((1, 1), z),
            pl.BlockSpec((1, 1), z),
            pl.BlockSpec((1, 1), z),
            pl.BlockSpec((1, 1), z),
            pl.BlockSpec((1, 1), z),
            pl.BlockSpec((1, 1), z),
            pl.BlockSpec((1, 1), z),
            pl.BlockSpec((MAXN - 1, OUT), z),
            pl.BlockSpec((OUT, OUT), z),
            pl.BlockSpec((NC, OUT), z),
            pl.BlockSpec((1, NC), z),
        ],
        out_specs=pl.BlockSpec((1, NC), z),
        out_shape=jax.ShapeDtypeStruct((1, NC), jnp.float32),
        scratch_shapes=[pltpu.VMEM((1, OUT), jnp.float32),
                        pltpu.VMEM((1, 128), jnp.float32)],
    )(rowt, col, h, aggp, dmisc_f, crb, ccb, bn1g, bn1b, bn2g, bn2b,
      gcb, lw9, lw128, clsw, clsb)


def kernel(h, edge_index, fc_w, attn_w, convrow_w, convrow_b, bn1_g, bn1_b,
           convcol_w, convcol_b, bn2_g, bn2_b, gc_w, gc_b, localw, cls_w,
           cls_b):
    src = edge_index[0]
    dst = edge_index[1]

    a1 = attn_w[:OUT, 0]
    a2 = attn_w[OUT:, 0]
    w0 = convrow_w[0, 0, 0, :]
    w1 = convrow_w[0, 0, 1, :]
    zero = jnp.zeros((OUT,), jnp.float32)
    bmat = jnp.stack([a1, a2, w0, w1, zero, zero, zero, zero], axis=1)
    cmat = jnp.stack([zero, zero, zero, zero, gc_w[:, 0], zero, zero, zero],
                     axis=1)

    z, aux = _tc0(h, fc_w, bmat, cmat)
    za1 = aux[:, 0]
    za2 = aux[:, 1]
    zw0 = aux[:, 2]
    zw1 = aux[:, 3]
    hw_row = aux[:, 4].reshape(1, N)

    histd_flat, hists_flat = _sc1(src, dst)
    offs, dmisc_i, dmisc_f = _tc1(histd_flat.reshape(TILES, N),
                                  hists_flat.reshape(TILES, N), hw_row)

    nb, aggp_flat = _sc2(src, dst, offs.reshape(-1), dmisc_f[0])

    cc = jnp.pad(convcol_w[0, 0, :, 0], (0, 16 - MAXN))
    rowt_flat, idx_flat, cw_flat = _sc3a(za1, za2, zw0, zw1,
                                         dmisc_i[0], dmisc_i[1], nb, cc)

    col = _sc3b(z, idx_flat, cw_flat)

    rowt = rowt_flat.reshape(MAXN - 1, N)
    aggp = aggp_flat.reshape(TILES, N)

    crb = convrow_b.reshape(1, 1)
    ccb = convcol_b.reshape(1, 1)
    out = _tc3(rowt, col, h, aggp, dmisc_f, crb, ccb,
               bn1_g.reshape(1, 1), bn1_b.reshape(1, 1),
               bn2_g.reshape(1, 1), bn2_b.reshape(1, 1),
               gc_b.reshape(1, 1), localw[:MAXN - 1], localw[MAXN - 1:],
               cls_w, cls_b.reshape(1, NC))
    return out


# speedup vs baseline: 1.0261x; 1.0169x over previous
import jax
import jax.numpy as jnp
from jax import lax
from jax.experimental import pallas as pl
from jax.experimental.pallas import tpu as pltpu
from jax.experimental.pallas import tpu_sc as plsc

N = 25600
E = 409600
IN = 128
OUT = 128
NC = 40
MAXN = 10

NCORE = 2
NSUB = 16
TILES = NCORE * NSUB
EPT = E // TILES
NPT = N // TILES
CHUNK = 128
NCH = EPT // CHUNK
GN = 8
GROWS = GN * MAXN
NGCH = NPT // GN
NBSZ = N * MAXN + TILES * CHUNK

_MESH = plsc.VectorSubcoreMesh(core_axis_name="c", subcore_axis_name="s",
                               num_cores=NCORE, num_subcores=NSUB)
_SC_PARAMS = pltpu.CompilerParams(needs_layout_passes=False)


def _wid():
    return lax.axis_index("s") * NCORE + lax.axis_index("c")


def _tc0_body(h_ref, fcw_ref, bmat_ref, cmat_ref, z_ref, aux_ref):
    h = h_ref[...]
    z = jnp.dot(h, fcw_ref[...], preferred_element_type=jnp.float32)
    z_ref[...] = z
    aux = (jnp.dot(z, bmat_ref[...], preferred_element_type=jnp.float32)
           + jnp.dot(h, cmat_ref[...], preferred_element_type=jnp.float32))
    aux_ref[...] = aux


def _tc0(h, fc_w, bmat, cmat):
    blk = 1600
    grid = (N // blk,)
    return pl.pallas_call(
        _tc0_body,
        grid=grid,
        in_specs=[
            pl.BlockSpec((blk, IN), lambda i: (i, 0)),
            pl.BlockSpec((IN, OUT), lambda i: (0, 0)),
            pl.BlockSpec((OUT, 8), lambda i: (0, 0)),
            pl.BlockSpec((IN, 8), lambda i: (0, 0)),
        ],
        out_specs=[
            pl.BlockSpec((blk, OUT), lambda i: (i, 0)),
            pl.BlockSpec((blk, 8), lambda i: (i, 0)),
        ],
        out_shape=[
            jax.ShapeDtypeStruct((N, OUT), jnp.float32),
            jax.ShapeDtypeStruct((N, 8), jnp.float32),
        ],
    )(h, fc_w, bmat, cmat)


def _sc1_body(src_hbm, dst_hbm, histd_out, hists_out, src_v, dst_v, hd_v, hs_v):
    w = _wid()
    pltpu.sync_copy(src_hbm.at[pl.ds(w * EPT, EPT)], src_v)
    pltpu.sync_copy(dst_hbm.at[pl.ds(w * EPT, EPT)], dst_v)

    zeros16 = jnp.zeros((16,), jnp.int32)

    def zbody(i, _):
        hd_v[pl.ds(i * 16, 16)] = zeros16
        hs_v[pl.ds(i * 16, 16)] = zeros16
        return 0

    lax.fori_loop(0, N // 16, zbody, 0)

    def body(v, _):
        dv = dst_v[pl.ds(v * 16, 16)]
        cnt, last = plsc.scan_count(dv)
        base = plsc.load_gather(hd_v, [dv])
        plsc.store_scatter(hd_v, [dv], base + cnt, mask=last)
        sv = src_v[pl.ds(v * 16, 16)]
        cnt2, last2 = plsc.scan_count(sv)
        base2 = plsc.load_gather(hs_v, [sv])
        plsc.store_scatter(hs_v, [sv], base2 + cnt2, mask=last2)
        return 0

    lax.fori_loop(0, EPT // 16, body, 0)
    pltpu.sync_copy(hd_v, histd_out.at[pl.ds(w * N, N)])
    pltpu.sync_copy(hs_v, hists_out.at[pl.ds(w * N, N)])


def _sc1(src, dst):
    f = pl.kernel(
        _sc1_body,
        out_type=(jax.ShapeDtypeStruct((TILES * N,), jnp.int32),
                  jax.ShapeDtypeStruct((TILES * N,), jnp.int32)),
        mesh=_MESH,
        compiler_params=_SC_PARAMS,
        scratch_types=[
            pltpu.VMEM((EPT,), jnp.int32),
            pltpu.VMEM((EPT,), jnp.int32),
            pltpu.VMEM((N,), jnp.int32),
            pltpu.VMEM((N,), jnp.int32),
        ],
    )
    return f(src, dst)


def _tc1_body(histd_ref, hists_ref, hw_ref, offs_ref, di_ref, df_ref):
    hd = histd_ref[...]
    run = jnp.zeros_like(hd[0:1, :])
    rows = []
    for t in range(TILES):
        rows.append(run)
        run = run + hd[t:t + 1, :]
    offs_ref[...] = jnp.concatenate(rows, axis=0)
    deg_in = run
    deg_out = jnp.sum(hists_ref[...], axis=0, keepdims=True)
    nsrc = lax.rsqrt(jnp.maximum(deg_out, 1).astype(jnp.float32))
    nd = lax.rsqrt(jnp.maximum(deg_in, 1).astype(jnp.float32))
    msgval = hw_ref[...] * nsrc
    min_deg = jnp.minimum(jnp.maximum(deg_in, 1), MAXN)
    di_ref[...] = jnp.concatenate([deg_in, min_deg], axis=0)
    df_ref[...] = jnp.concatenate([msgval, nd], axis=0)


def _tc1(histd, hists, hw_row):
    blk = 3200
    grid = (N // blk,)
    return pl.pallas_call(
        _tc1_body,
        grid=grid,
        in_specs=[
            pl.BlockSpec((TILES, blk), lambda i: (0, i)),
            pl.BlockSpec((TILES, blk), lambda i: (0, i)),
            pl.BlockSpec((1, blk), lambda i: (0, i)),
        ],
        out_specs=[
            pl.BlockSpec((TILES, blk), lambda i: (0, i)),
            pl.BlockSpec((2, blk), lambda i: (0, i)),
            pl.BlockSpec((2, blk), lambda i: (0, i)),
        ],
        out_shape=[
            jax.ShapeDtypeStruct((TILES, N), jnp.int32),
            jax.ShapeDtypeStruct((2, N), jnp.int32),
            jax.ShapeDtypeStruct((2, N), jnp.float32),
        ],
    )(histd, hists, hw_row)


ZC = NBSZ // NSUB
ZB = 2032


def _sc2_body(src_hbm, dst_hbm, offs_hbm, msg_hbm, nb_out, aggp_out,
              src_v, dst_v, cnt_v, msg_v, agg_v, widx_v, widxB_v, zero_v,
              nb_sh, semA, semB):
    s = lax.axis_index("s")
    core = lax.axis_index("c")
    w = s * NCORE + core
    pltpu.sync_copy(src_hbm.at[pl.ds(w * EPT, EPT)], src_v)
    pltpu.sync_copy(dst_hbm.at[pl.ds(w * EPT, EPT)], dst_v)
    pltpu.sync_copy(offs_hbm.at[pl.ds(w * N, N)], cnt_v)
    pltpu.sync_copy(msg_hbm, msg_v)

    zeros16f = jnp.zeros((16,), jnp.float32)
    zeros16 = jnp.zeros((16,), jnp.int32)

    def zbody(i, _):
        agg_v[pl.ds(i * 16, 16)] = zeros16f
        return 0

    lax.fori_loop(0, N // 16, zbody, 0)

    def zbody2(i, _):
        zero_v[pl.ds(i * 16, 16)] = zeros16
        return 0

    lax.fori_loop(0, ZB // 16, zbody2, 0)
    for i in range(ZC // ZB):
        pltpu.sync_copy(zero_v, nb_sh.at[pl.ds(s * ZC + i * ZB, ZB)])
    plsc.subcore_barrier()

    iota16 = lax.iota(jnp.int32, 16)

    def chunk_work(p, cr, widx_b, sem):
        @pl.when(p > 0)
        def _():
            pltpu.make_async_copy(src_v.at[pl.ds((cr - 2) * CHUNK, CHUNK)],
                                  nb_sh.at[widx_b], sem).wait()

        for k in range(CHUNK // 16):
            dv = dst_v[pl.ds(cr * CHUNK + k * 16, 16)]
            cntv, lastv = plsc.scan_count(dv)
            rank_i = cntv - 1
            base = plsc.load_gather(cnt_v, [dv])
            plsc.store_scatter(cnt_v, [dv], base + cntv, mask=lastv)
            rank = base + rank_i
            ok = rank < MAXN
            dump = N * MAXN + w * CHUNK + k * 16 + iota16
            widx = jnp.where(ok, dv * MAXN + rank, dump)
            widx_b[pl.ds(k * 16, 16)] = widx
            sv = src_v[pl.ds(cr * CHUNK + k * 16, 16)]
            mv = plsc.load_gather(msg_v, [sv])
            nround = jnp.max(cntv)

            def rbody(r, _):
                plsc.addupdate_scatter(agg_v, [dv], mv, mask=(rank_i == r))
                return 0

            lax.fori_loop(0, nround, rbody, 0)
        pltpu.async_copy(src_v.at[pl.ds(cr * CHUNK, CHUNK)],
                         nb_sh.at[widx_b], sem, add=True)

    def body(p, _):
        chunk_work(p, p * 2, widx_v, semA)
        chunk_work(p, p * 2 + 1, widxB_v, semB)
        return 0

    lax.fori_loop(0, NCH // 2, body, 0)
    pltpu.make_async_copy(src_v.at[pl.ds((NCH - 2) * CHUNK, CHUNK)],
                          nb_sh.at[widx_v], semA).wait()
    pltpu.make_async_copy(src_v.at[pl.ds((NCH - 1) * CHUNK, CHUNK)],
                          nb_sh.at[widxB_v], semB).wait()
    pltpu.sync_copy(agg_v, aggp_out.at[pl.ds(w * N, N)])
    plsc.subcore_barrier()

    @pl.when(s == 0)
    def _():
        pltpu.sync_copy(nb_sh, nb_out.at[pl.ds(core * NBSZ, NBSZ)])


def _sc2(src, dst, offs_flat, msgval):
    f = pl.kernel(
        _sc2_body,
        out_type=(jax.ShapeDtypeStruct((NCORE * NBSZ,), jnp.int32),
                  jax.ShapeDtypeStruct((TILES * N,), jnp.float32)),
        mesh=_MESH,
        compiler_params=_SC_PARAMS,
        scratch_types=[
            pltpu.VMEM((EPT,), jnp.int32),
            pltpu.VMEM((EPT,), jnp.int32),
            pltpu.VMEM((N,), jnp.int32),
            pltpu.VMEM((N,), jnp.float32),
            pltpu.VMEM((N,), jnp.float32),
            pltpu.VMEM((CHUNK,), jnp.int32),
            pltpu.VMEM((CHUNK,), jnp.int32),
            pltpu.VMEM((ZB,), jnp.int32),
            pltpu.VMEM_SHARED((NBSZ,), jnp.int32),
            pltpu.SemaphoreType.DMA,
            pltpu.SemaphoreType.DMA,
        ],
    )
    return f(src, dst, offs_flat, msgval)


def _sc3a_body(za1_hbm, za2_hbm, zw0_hbm, zw1_hbm, deg_hbm, mind_hbm,
               nb_hbm, cc_hbm, rowt_out, idx_out, cw_out,
               za1_v, zw0_v, zw1_v, za2_v, deg_v, mind_v, nbl_v, nblb_v,
               row_v, idx_v, cw_v, cc_v):
    w = _wid()
    base = w * NPT
    pltpu.sync_copy(za1_hbm, za1_v)
    pltpu.sync_copy(zw0_hbm, zw0_v)
    pltpu.sync_copy(zw1_hbm, zw1_v)
    pltpu.sync_copy(za2_hbm.at[pl.ds(base, NPT)], za2_v)
    pltpu.sync_copy(deg_hbm.at[pl.ds(base, NPT)], deg_v)
    pltpu.sync_copy(mind_hbm.at[pl.ds(base, NPT)], mind_v)
    pltpu.sync_copy(nb_hbm.at[pl.ds(base * MAXN, NPT * MAXN)], nbl_v)
    pltpu.sync_copy(nb_hbm.at[pl.ds(NBSZ + base * MAXN, NPT * MAXN)], nblb_v)
    pltpu.sync_copy(cc_hbm, cc_v)

    def mbody(i, _):
        nbl_v[pl.ds(i * 16, 16)] = (nbl_v[pl.ds(i * 16, 16)]
                                    + nblb_v[pl.ds(i * 16, 16)])
        return 0

    lax.fori_loop(0, NPT * MAXN // 16, mbody, 0)

    iota16 = lax.iota(jnp.int32, 16)
    cc_all = cc_v[...]

    def body(g, _):
        loc = g * 16 + iota16
        nabs = base + loc
        degv = deg_v[pl.ds(g * 16, 16)]
        mdv = mind_v[pl.ds(g * 16, 16)]
        za2v = za2_v[pl.ds(g * 16, 16)]
        isolated = degv <= 0

        nbs = []
        es = []
        for j in range(MAXN):
            jj = lax.rem(jnp.full((16,), j, jnp.int32), mdv)
            nbj = plsc.load_gather(nbl_v, [loc * MAXN + jj])
            nbj = jnp.where(isolated, nabs, nbj)
            nbs.append(nbj)
            t = plsc.load_gather(za1_v, [nbj]) + za2v
            es.append(jnp.where(t >= 0, t, 0.01 * t))
        mx = es[0]
        for j in range(1, MAXN):
            mx = jnp.maximum(mx, es[j])
        exs = [jnp.exp(es[j] - mx) for j in range(MAXN)]
        s = exs[0]
        for j in range(1, MAXN):
            s = s + exs[j]
        inv = 1.0 / s
        alphas = [exs[j] * inv for j in range(MAXN)]

        w0g = [plsc.load_gather(zw0_v, [nbs[j]]) for j in range(MAXN - 1)]
        w1g = [None] + [plsc.load_gather(zw1_v, [nbs[j]]) for j in range(1, MAXN)]
        for i in range(MAXN - 1):
            row_v[pl.ds(i * NPT + g * 16, 16)] = (alphas[i] * w0g[i]
                                                  + alphas[i + 1] * w1g[i + 1])
        for j in range(MAXN):
            sidx = loc * MAXN + j
            plsc.store_scatter(idx_v, [sidx], nbs[j])
            plsc.store_scatter(cw_v, [sidx], alphas[j] * cc_all[j])
        return 0

    lax.fori_loop(0, NPT // 16, body, 0)
    for i in range(MAXN - 1):
        pltpu.sync_copy(row_v.at[pl.ds(i * NPT, NPT)],
                        rowt_out.at[pl.ds(i * N + base, NPT)])
    pltpu.sync_copy(idx_v, idx_out.at[pl.ds(base * MAXN, NPT * MAXN)])
    pltpu.sync_copy(cw_v, cw_out.at[pl.ds(base * MAXN, NPT * MAXN)])


def _sc3a(za1, za2, zw0, zw1, deg, mind, nb, cc):
    f = pl.kernel(
        _sc3a_body,
        out_type=(jax.ShapeDtypeStruct(((MAXN - 1) * N,), jnp.float32),
                  jax.ShapeDtypeStruct((N * MAXN,), jnp.int32),
                  jax.ShapeDtypeStruct((N * MAXN,), jnp.float32)),
        mesh=_MESH,
        compiler_params=_SC_PARAMS,
        scratch_types=[
            pltpu.VMEM((N,), jnp.float32),
            pltpu.VMEM((N,), jnp.float32),
            pltpu.VMEM((N,), jnp.float32),
            pltpu.VMEM((NPT,), jnp.float32),
            pltpu.VMEM((NPT,), jnp.int32),
            pltpu.VMEM((NPT,), jnp.int32),
            pltpu.VMEM((NPT * MAXN,), jnp.int32),
            pltpu.VMEM((NPT * MAXN,), jnp.int32),
            pltpu.VMEM(((MAXN - 1) * NPT,), jnp.float32),
            pltpu.VMEM((NPT * MAXN,), jnp.int32),
            pltpu.VMEM((NPT * MAXN,), jnp.float32),
            pltpu.VMEM((16,), jnp.float32),
        ],
    )
    return f(za1, za2, zw0, zw1, deg, mind, nb, cc)


def _sc3b_body(z_hbm, idx_hbm, cw_hbm, col_out,
               idx_v, cw_v, zb0_v, zb1_v, colstA_v, colstB_v,
               sem0, sem1, semo):
    w = _wid()
    eb = w * NPT * MAXN
    pltpu.sync_copy(idx_hbm.at[pl.ds(eb, NPT * MAXN)], idx_v)
    pltpu.sync_copy(cw_hbm.at[pl.ds(eb, NPT * MAXN)], cw_v)

    def compute(c, zbuf_v, colst_v):
        wv = [cw_v[pl.ds(c * GROWS + t * 16, 16)] for t in range(GROWS // 16)]
        for nl in range(GN):
            accs = [jnp.zeros((16,), jnp.float32) for _ in range(OUT // 16)]
            for j in range(MAXN):
                r = nl * MAXN + j
                wgt = wv[r // 16][r % 16]
                for q in range(OUT // 16):
                    accs[q] = accs[q] + zbuf_v[r, pl.ds(q * 16, 16)] * wgt
            for q in range(OUT // 16):
                colst_v[nl, pl.ds(q * 16, 16)] = accs[q]

    def fire(c, zbuf_v, sem):
        pltpu.async_copy(z_hbm.at[idx_v.at[pl.ds(c * GROWS, GROWS)]],
                         zbuf_v, sem)

    def drain(c, zbuf_v, sem):
        pltpu.make_async_copy(z_hbm.at[idx_v.at[pl.ds(c * GROWS, GROWS)]],
                              zbuf_v, sem).wait()

    def out_slice(c):
        return col_out.at[pl.ds(w * NPT + c * GN, GN)]

    fire(0, zb0_v, sem0)
    fire(1, zb1_v, sem1)

    def half(p, c, zbuf_v, gsem, colst_v):
        drain(c, zbuf_v, gsem)

        @pl.when(p > 0)
        def _():
            pltpu.make_async_copy(colst_v, out_slice(c - 2), semo).wait()

        compute(c, zbuf_v, colst_v)
        pltpu.async_copy(colst_v, out_slice(c), semo)

        @pl.when(c + 2 < NGCH)
        def _():
            fire(c + 2, zbuf_v, gsem)

    def body(p, _):
        c0 = p * 2
        half(p, c0, zb0_v, sem0, colstA_v)
        half(p, c0 + 1, zb1_v, sem1, colstB_v)
        return 0

    lax.fori_loop(0, NGCH // 2, body, 0)
    pltpu.make_async_copy(colstA_v, out_slice(NGCH - 2), semo).wait()
    pltpu.make_async_copy(colstB_v, out_slice(NGCH - 1), semo).wait()


def _sc3b(z, idx_flat, cw_flat):
    f = pl.kernel(
        _sc3b_body,
        out_type=jax.ShapeDtypeStruct((N, OUT), jnp.float32),
        mesh=_MESH,
        compiler_params=_SC_PARAMS,
        scratch_types=[
            pltpu.VMEM((NPT * MAXN,), jnp.int32),
            pltpu.VMEM((NPT * MAXN,), jnp.float32),
            pltpu.VMEM((GROWS, OUT), jnp.float32),
            pltpu.VMEM((GROWS, OUT), jnp.float32),
            pltpu.VMEM((GN, OUT), jnp.float32),
            pltpu.VMEM((GN, OUT), jnp.float32),
            pltpu.SemaphoreType.DMA,
            pltpu.SemaphoreType.DMA,
            pltpu.SemaphoreType.DMA,
        ],
    )
    return f(z, idx_flat, cw_flat)


def _tc3_body(rowt_ref, col_ref, h_ref, aggp_ref, df_ref,
              crb_ref, ccb_ref, bn1g_ref, bn1b_ref, bn2g_ref, bn2b_ref,
              gcb_ref, lw9_ref, lw128_ref, clsw_ref, clsb_ref,
              out_ref, acc_ref, st_ref):
    i = pl.program_id(0)
    nblk = pl.num_programs(0) // 2
    blk = col_ref.shape[0]

    r = rowt_ref[...] + crb_ref[0, 0]
    c = col_ref[...] + ccb_ref[0, 0]

    @pl.when(i < nblk)
    def _():
        stats = jnp.concatenate(
            [x.reshape(1, 1) for x in
             (jnp.sum(r), jnp.sum(r * r), jnp.sum(c), jnp.sum(c * c))]
            + [jnp.zeros((1, 124), jnp.float32)], axis=1)
        st_ref[...] = jnp.where(i == 0, stats, st_ref[...] + stats)

    @pl.when(i >= nblk)
    def _():
        st = st_ref[...]
        nr = float(N * (MAXN - 1))
        ncl = float(N * OUT)
        mu1 = st[0, 0] / nr
        var1 = st[0, 1] / nr - mu1 * mu1
        mu2 = st[0, 2] / ncl
        var2 = st[0, 3] / ncl - mu2 * mu2
        inv1 = bn1g_ref[0, 0] * lax.rsqrt(var1 + 1e-5)
        inv2 = bn2g_ref[0, 0] * lax.rsqrt(var2 + 1e-5)

        r1 = jnp.maximum((r - mu1) * inv1 + bn1b_ref[0, 0], 0.0)
        c1 = jnp.maximum((c - mu2) * inv2 + bn2b_ref[0, 0], 0.0)

        gat9 = lax.dot_general(r1, lw9_ref[...], (((0,), (0,)), ((), ())),
                               preferred_element_type=jnp.float32)
        gatc = jnp.dot(c1, lw128_ref[...], preferred_element_type=jnp.float32)
        feats = jnp.maximum(gat9 + gatc + h_ref[...], 0.0)

        agg = jnp.sum(aggp_ref[...], axis=0, keepdims=True)
        gc = agg * df_ref[1:2, :] + gcb_ref[0, 0]
        ng = blk // IN
        gcr = jnp.concatenate([gc[:, k * IN:(k + 1) * IN] for k in range(ng)],
                              axis=0)
        gcr = gcr - jnp.max(gcr, axis=1, keepdims=True)
        egc = jnp.exp(gcr)
        gw = egc / jnp.sum(egc, axis=1, keepdims=True)
        contrib = jnp.zeros((1, OUT), jnp.float32)
        for k in range(ng):
            contrib = contrib + jnp.dot(gw[k:k + 1, :],
                                        feats[k * IN:(k + 1) * IN, :],
                                        preferred_element_type=jnp.float32)

        newacc = jnp.where(i == nblk, contrib, acc_ref[...] + contrib)
        acc_ref[...] = newacc

        @pl.when(i == 2 * nblk - 1)
        def _():
            hg = newacc / float(N)
            out_ref[...] = lax.dot_general(
                hg, clsw_ref[...], (((1,), (1,)), ((), ())),
                preferred_element_type=jnp.float32) + clsb_ref[...]


def _tc3(rowt, col, h, aggp, dmisc_f, crb, ccb, bn1g, bn1b, bn2g,
         bn2b, gcb, lw9, lw128, clsw, clsb):
    blk = 3200
    nblk = N // blk
    grid = (2 * nblk,)
    bi = lambda i: (i % nblk, 0)
    bj = lambda i: (0, i % nblk)
    z = lambda i: (0, 0)
    return pl.pallas_call(
        _tc3_body,
        grid=grid,
        in_specs=[
            pl.BlockSpec((MAXN - 1, blk), bj),
            pl.BlockSpec((blk, OUT), bi),
            pl.BlockSpec((blk, IN), bi),
            pl.BlockSpec((TILES, blk), bj),
            pl.BlockSpec((2, blk), bj),
            pl.BlockSpec((1, 1), z),
            pl.BlockSpec((1, 1), z),
            pl.BlockSpec((1, 1), z),
            pl.BlockSpec((1, 1), z),
            pl.BlockSpec((1, 1), z),
            pl.BlockSpec((1, 1), z),
            pl.BlockSpec((1, 1), z),
            pl.BlockSpec((MAXN - 1, OUT), z),
            pl.BlockSpec((OUT, OUT), z),
            pl.BlockSpec((NC, OUT), z),
            pl.BlockSpec((1, NC), z),
        ],
        out_specs=pl.BlockSpec((1, NC), z),
        out_shape=jax.ShapeDtypeStruct((1, NC), jnp.float32),
        scratch_shapes=[pltpu.VMEM((1, OUT), jnp.float32),
                        pltpu.VMEM((1, 128), jnp.float32)],
    )(rowt, col, h, aggp, dmisc_f, crb, ccb, bn1g, bn1b, bn2g, bn2b,
      gcb, lw9, lw128, clsw, clsb)


def kernel(h, edge_index, fc_w, attn_w, convrow_w, convrow_b, bn1_g, bn1_b,
           convcol_w, convcol_b, bn2_g, bn2_b, gc_w, gc_b, localw, cls_w,
           cls_b):
    src = edge_index[0]
    dst = edge_index[1]

    a1 = attn_w[:OUT, 0]
    a2 = attn_w[OUT:, 0]
    w0 = convrow_w[0, 0, 0, :]
    w1 = convrow_w[0, 0, 1, :]
    zero = jnp.zeros((OUT,), jnp.float32)
    bmat = jnp.stack([a1, a2, w0, w1, zero, zero, zero, zero], axis=1)
    cmat = jnp.stack([zero, zero, zero, zero, gc_w[:, 0], zero, zero, zero],
                     axis=1)

    z, aux = _tc0(h, fc_w, bmat, cmat)
    za1 = aux[:, 0]
    za2 = aux[:, 1]
    zw0 = aux[:, 2]
    zw1 = aux[:, 3]
    hw_row = aux[:, 4].reshape(1, N)

    histd_flat, hists_flat = _sc1(src, dst)
    offs, dmisc_i, dmisc_f = _tc1(histd_flat.reshape(TILES, N),
                                  hists_flat.reshape(TILES, N), hw_row)

    nb, aggp_flat = _sc2(src, dst, offs.reshape(-1), dmisc_f[0])

    cc = jnp.pad(convcol_w[0, 0, :, 0], (0, 16 - MAXN))
    rowt_flat, idx_flat, cw_flat = _sc3a(za1, za2, zw0, zw1,
                                         dmisc_i[0], dmisc_i[1], nb, cc)

    col = _sc3b(z, idx_flat, cw_flat)

    rowt = rowt_flat.reshape(MAXN - 1, N)
    aggp = aggp_flat.reshape(TILES, N)

    crb = convrow_b.reshape(1, 1)
    ccb = convcol_b.reshape(1, 1)
    out = _tc3(rowt, col, h, aggp, dmisc_f, crb, ccb,
               bn1_g.reshape(1, 1), bn1_b.reshape(1, 1),
               bn2_g.reshape(1, 1), bn2_b.reshape(1, 1),
               gc_b.reshape(1, 1), localw[:MAXN - 1], localw[MAXN - 1:],
               cls_w, cls_b.reshape(1, NC))
    return out


# speedup vs baseline: 1.0948x; 1.0670x over previous
import jax
import jax.numpy as jnp
from jax import lax
from jax.experimental import pallas as pl
from jax.experimental.pallas import tpu as pltpu
from jax.experimental.pallas import tpu_sc as plsc

N = 25600
E = 409600
IN = 128
OUT = 128
NC = 40
MAXN = 10

NCORE = 2
NSUB = 16
TILES = NCORE * NSUB
EPT = E // TILES
NPT = N // TILES
CHUNK = 128
NCH = EPT // CHUNK
GN = 8
GROWS = GN * MAXN
NGCH = NPT // GN
NBSZ = N * MAXN + TILES * CHUNK

_MESH = plsc.VectorSubcoreMesh(core_axis_name="c", subcore_axis_name="s",
                               num_cores=NCORE, num_subcores=NSUB)
_SC_PARAMS = pltpu.CompilerParams(needs_layout_passes=False)


def _wid():
    return lax.axis_index("s") * NCORE + lax.axis_index("c")


def _tc0_body(h_ref, fcw_ref, bmat_ref, cmat_ref, z_ref, aux_ref):
    h = h_ref[...]
    z = jnp.dot(h, fcw_ref[...], preferred_element_type=jnp.float32)
    z_ref[...] = z
    aux = (lax.dot_general(bmat_ref[...], z, (((0,), (1,)), ((), ())),
                           preferred_element_type=jnp.float32)
           + lax.dot_general(cmat_ref[...], h, (((0,), (1,)), ((), ())),
                             preferred_element_type=jnp.float32))
    aux_ref[...] = aux


def _tc0(h, fc_w, bmat, cmat):
    blk = 3200
    grid = (N // blk,)
    return pl.pallas_call(
        _tc0_body,
        grid=grid,
        in_specs=[
            pl.BlockSpec((blk, IN), lambda i: (i, 0)),
            pl.BlockSpec((IN, OUT), lambda i: (0, 0)),
            pl.BlockSpec((OUT, 8), lambda i: (0, 0)),
            pl.BlockSpec((IN, 8), lambda i: (0, 0)),
        ],
        out_specs=[
            pl.BlockSpec((blk, OUT), lambda i: (i, 0)),
            pl.BlockSpec((8, blk), lambda i: (0, i)),
        ],
        out_shape=[
            jax.ShapeDtypeStruct((N, OUT), jnp.float32),
            jax.ShapeDtypeStruct((8, N), jnp.float32),
        ],
    )(h, fc_w, bmat, cmat)


def _sc1_body(src_hbm, dst_hbm, histd_out, hists_out, src_v, dst_v, hd_v, hs_v):
    w = _wid()
    pltpu.sync_copy(src_hbm.at[pl.ds(w * EPT, EPT)], src_v)
    pltpu.sync_copy(dst_hbm.at[pl.ds(w * EPT, EPT)], dst_v)

    zeros16 = jnp.zeros((16,), jnp.int32)

    def zbody(i, _):
        hd_v[pl.ds(i * 16, 16)] = zeros16
        hs_v[pl.ds(i * 16, 16)] = zeros16
        return 0

    lax.fori_loop(0, N // 16, zbody, 0)

    def body(v, _):
        dv = dst_v[pl.ds(v * 16, 16)]
        cnt, last = plsc.scan_count(dv)
        base = plsc.load_gather(hd_v, [dv])
        plsc.store_scatter(hd_v, [dv], base + cnt, mask=last)
        sv = src_v[pl.ds(v * 16, 16)]
        cnt2, last2 = plsc.scan_count(sv)
        base2 = plsc.load_gather(hs_v, [sv])
        plsc.store_scatter(hs_v, [sv], base2 + cnt2, mask=last2)
        return 0

    lax.fori_loop(0, EPT // 16, body, 0)
    pltpu.sync_copy(hd_v, histd_out.at[pl.ds(w * N, N)])
    pltpu.sync_copy(hs_v, hists_out.at[pl.ds(w * N, N)])


def _sc1(src, dst):
    f = pl.kernel(
        _sc1_body,
        out_type=(jax.ShapeDtypeStruct((TILES * N,), jnp.int32),
                  jax.ShapeDtypeStruct((TILES * N,), jnp.int32)),
        mesh=_MESH,
        compiler_params=_SC_PARAMS,
        scratch_types=[
            pltpu.VMEM((EPT,), jnp.int32),
            pltpu.VMEM((EPT,), jnp.int32),
            pltpu.VMEM((N,), jnp.int32),
            pltpu.VMEM((N,), jnp.int32),
        ],
    )
    return f(src, dst)


def _tc1_body(histd_ref, hists_ref, aux_ref, offs_ref, di_ref, df_ref):
    hw_ref = aux_ref[4:5, :]
    hd = histd_ref[...]
    run = jnp.zeros_like(hd[0:1, :])
    rows = []
    for t in range(TILES):
        rows.append(run)
        run = run + hd[t:t + 1, :]
    offs_ref[...] = jnp.concatenate(rows, axis=0)
    deg_in = run
    deg_out = jnp.sum(hists_ref[...], axis=0, keepdims=True)
    nsrc = lax.rsqrt(jnp.maximum(deg_out, 1).astype(jnp.float32))
    nd = lax.rsqrt(jnp.maximum(deg_in, 1).astype(jnp.float32))
    msgval = hw_ref * nsrc
    min_deg = jnp.minimum(jnp.maximum(deg_in, 1), MAXN)
    di_ref[...] = jnp.concatenate([deg_in, min_deg], axis=0)
    df_ref[...] = jnp.concatenate([msgval, nd], axis=0)


def _tc1(histd, hists, aux_t):
    blk = 3200
    grid = (N // blk,)
    return pl.pallas_call(
        _tc1_body,
        grid=grid,
        in_specs=[
            pl.BlockSpec((TILES, blk), lambda i: (0, i)),
            pl.BlockSpec((TILES, blk), lambda i: (0, i)),
            pl.BlockSpec((8, blk), lambda i: (0, i)),
        ],
        out_specs=[
            pl.BlockSpec((TILES, blk), lambda i: (0, i)),
            pl.BlockSpec((2, blk), lambda i: (0, i)),
            pl.BlockSpec((2, blk), lambda i: (0, i)),
        ],
        out_shape=[
            jax.ShapeDtypeStruct((TILES, N), jnp.int32),
            jax.ShapeDtypeStruct((2, N), jnp.int32),
            jax.ShapeDtypeStruct((2, N), jnp.float32),
        ],
    )(histd, hists, aux_t)


ZC = NBSZ // NSUB
ZB = 2032


def _sc2_body(src_hbm, dst_hbm, offs_hbm, msg_hbm, nb_out, aggp_out,
              src_v, dst_v, cnt_v, msg_v, agg_v, widx_v, widxB_v, zero_v,
              nb_sh, semA, semB):
    s = lax.axis_index("s")
    core = lax.axis_index("c")
    w = s * NCORE + core
    pltpu.sync_copy(src_hbm.at[pl.ds(w * EPT, EPT)], src_v)
    pltpu.sync_copy(dst_hbm.at[pl.ds(w * EPT, EPT)], dst_v)
    pltpu.sync_copy(offs_hbm.at[pl.ds(w * N, N)], cnt_v)
    pltpu.sync_copy(msg_hbm.at[pl.ds(0, N)], msg_v)

    zeros16f = jnp.zeros((16,), jnp.float32)
    zeros16 = jnp.zeros((16,), jnp.int32)

    def zbody(i, _):
        agg_v[pl.ds(i * 16, 16)] = zeros16f
        return 0

    lax.fori_loop(0, N // 16, zbody, 0)

    def zbody2(i, _):
        zero_v[pl.ds(i * 16, 16)] = zeros16
        return 0

    lax.fori_loop(0, ZB // 16, zbody2, 0)
    for i in range(ZC // ZB):
        pltpu.sync_copy(zero_v, nb_sh.at[pl.ds(s * ZC + i * ZB, ZB)])
    plsc.subcore_barrier()

    iota16 = lax.iota(jnp.int32, 16)

    def chunk_work(p, cr, widx_b, sem):
        @pl.when(p > 0)
        def _():
            pltpu.make_async_copy(src_v.at[pl.ds((cr - 2) * CHUNK, CHUNK)],
                                  nb_sh.at[widx_b], sem).wait()

        for k in range(CHUNK // 16):
            dv = dst_v[pl.ds(cr * CHUNK + k * 16, 16)]
            cntv, lastv = plsc.scan_count(dv)
            rank_i = cntv - 1
            base = plsc.load_gather(cnt_v, [dv])
            plsc.store_scatter(cnt_v, [dv], base + cntv, mask=lastv)
            rank = base + rank_i
            ok = rank < MAXN
            dump = N * MAXN + w * CHUNK + k * 16 + iota16
            widx = jnp.where(ok, dv * MAXN + rank, dump)
            widx_b[pl.ds(k * 16, 16)] = widx
            sv = src_v[pl.ds(cr * CHUNK + k * 16, 16)]
            mv = plsc.load_gather(msg_v, [sv])
            nround = jnp.max(cntv)

            def rbody(r, _):
                plsc.addupdate_scatter(agg_v, [dv], mv, mask=(rank_i == r))
                return 0

            lax.fori_loop(0, nround, rbody, 0)
        pltpu.async_copy(src_v.at[pl.ds(cr * CHUNK, CHUNK)],
                         nb_sh.at[widx_b], sem, add=True)

    def body(p, _):
        chunk_work(p, p * 2, widx_v, semA)
        chunk_work(p, p * 2 + 1, widxB_v, semB)
        return 0

    lax.fori_loop(0, NCH // 2, body, 0)
    pltpu.make_async_copy(src_v.at[pl.ds((NCH - 2) * CHUNK, CHUNK)],
                          nb_sh.at[widx_v], semA).wait()
    pltpu.make_async_copy(src_v.at[pl.ds((NCH - 1) * CHUNK, CHUNK)],
                          nb_sh.at[widxB_v], semB).wait()
    pltpu.sync_copy(agg_v, aggp_out.at[pl.ds(w * N, N)])
    plsc.subcore_barrier()

    @pl.when(s == 0)
    def _():
        pltpu.sync_copy(nb_sh, nb_out.at[pl.ds(core * NBSZ, NBSZ)])


def _sc2(src, dst, offs_flat, msgval):
    f = pl.kernel(
        _sc2_body,
        out_type=(jax.ShapeDtypeStruct((NCORE * NBSZ,), jnp.int32),
                  jax.ShapeDtypeStruct((TILES * N,), jnp.float32)),
        mesh=_MESH,
        compiler_params=_SC_PARAMS,
        scratch_types=[
            pltpu.VMEM((EPT,), jnp.int32),
            pltpu.VMEM((EPT,), jnp.int32),
            pltpu.VMEM((N,), jnp.int32),
            pltpu.VMEM((N,), jnp.float32),
            pltpu.VMEM((N,), jnp.float32),
            pltpu.VMEM((CHUNK,), jnp.int32),
            pltpu.VMEM((CHUNK,), jnp.int32),
            pltpu.VMEM((ZB,), jnp.int32),
            pltpu.VMEM_SHARED((NBSZ,), jnp.int32),
            pltpu.SemaphoreType.DMA,
            pltpu.SemaphoreType.DMA,
        ],
    )
    return f(src, dst, offs_flat, msgval)


def _sc3a_body(aux_hbm, di_hbm, nb_hbm, cc_hbm, rowt_out, idx_out, cw_out,
               za1_v, zw0_v, zw1_v, za2_v, deg_v, mind_v, nbl_v, nblb_v,
               row_v, idx_v, cw_v, cc_v):
    w = _wid()
    base = w * NPT
    pltpu.sync_copy(aux_hbm.at[pl.ds(0, N)], za1_v)
    pltpu.sync_copy(aux_hbm.at[pl.ds(2 * N, N)], zw0_v)
    pltpu.sync_copy(aux_hbm.at[pl.ds(3 * N, N)], zw1_v)
    pltpu.sync_copy(aux_hbm.at[pl.ds(N + base, NPT)], za2_v)
    pltpu.sync_copy(di_hbm.at[pl.ds(base, NPT)], deg_v)
    pltpu.sync_copy(di_hbm.at[pl.ds(N + base, NPT)], mind_v)
    pltpu.sync_copy(nb_hbm.at[pl.ds(base * MAXN, NPT * MAXN)], nbl_v)
    pltpu.sync_copy(nb_hbm.at[pl.ds(NBSZ + base * MAXN, NPT * MAXN)], nblb_v)
    pltpu.sync_copy(cc_hbm, cc_v)

    def mbody(i, _):
        nbl_v[pl.ds(i * 16, 16)] = (nbl_v[pl.ds(i * 16, 16)]
                                    + nblb_v[pl.ds(i * 16, 16)])
        return 0

    lax.fori_loop(0, NPT * MAXN // 16, mbody, 0)

    iota16 = lax.iota(jnp.int32, 16)
    cc_all = cc_v[...]

    def body(g, _):
        loc = g * 16 + iota16
        nabs = base + loc
        degv = deg_v[pl.ds(g * 16, 16)]
        mdv = mind_v[pl.ds(g * 16, 16)]
        za2v = za2_v[pl.ds(g * 16, 16)]
        isolated = degv <= 0

        nbs = []
        es = []
        for j in range(MAXN):
            jj = lax.rem(jnp.full((16,), j, jnp.int32), mdv)
            nbj = plsc.load_gather(nbl_v, [loc * MAXN + jj])
            nbj = jnp.where(isolated, nabs, nbj)
            nbs.append(nbj)
            t = plsc.load_gather(za1_v, [nbj]) + za2v
            es.append(jnp.where(t >= 0, t, 0.01 * t))
        mx = es[0]
        for j in range(1, MAXN):
            mx = jnp.maximum(mx, es[j])
        exs = [jnp.exp(es[j] - mx) for j in range(MAXN)]
        s = exs[0]
        for j in range(1, MAXN):
            s = s + exs[j]
        inv = 1.0 / s
        alphas = [exs[j] * inv for j in range(MAXN)]

        w0g = [plsc.load_gather(zw0_v, [nbs[j]]) for j in range(MAXN - 1)]
        w1g = [None] + [plsc.load_gather(zw1_v, [nbs[j]]) for j in range(1, MAXN)]
        for i in range(MAXN - 1):
            row_v[pl.ds(i * NPT + g * 16, 16)] = (alphas[i] * w0g[i]
                                                  + alphas[i + 1] * w1g[i + 1])
        for j in range(MAXN):
            sidx = loc * MAXN + j
            plsc.store_scatter(idx_v, [sidx], nbs[j])
            plsc.store_scatter(cw_v, [sidx], alphas[j] * cc_all[j])
        return 0

    lax.fori_loop(0, NPT // 16, body, 0)
    for i in range(MAXN - 1):
        pltpu.sync_copy(row_v.at[pl.ds(i * NPT, NPT)],
                        rowt_out.at[pl.ds(i * N + base, NPT)])
    pltpu.sync_copy(idx_v, idx_out.at[pl.ds(base * MAXN, NPT * MAXN)])
    pltpu.sync_copy(cw_v, cw_out.at[pl.ds(base * MAXN, NPT * MAXN)])


def _sc3a(aux_flat, di_flat, nb, cc):
    f = pl.kernel(
        _sc3a_body,
        out_type=(jax.ShapeDtypeStruct(((MAXN - 1) * N,), jnp.float32),
                  jax.ShapeDtypeStruct((N * MAXN,), jnp.int32),
                  jax.ShapeDtypeStruct((N * MAXN,), jnp.float32)),
        mesh=_MESH,
        compiler_params=_SC_PARAMS,
        scratch_types=[
            pltpu.VMEM((N,), jnp.float32),
            pltpu.VMEM((N,), jnp.float32),
            pltpu.VMEM((N,), jnp.float32),
            pltpu.VMEM((NPT,), jnp.float32),
            pltpu.VMEM((NPT,), jnp.int32),
            pltpu.VMEM((NPT,), jnp.int32),
            pltpu.VMEM((NPT * MAXN,), jnp.int32),
            pltpu.VMEM((NPT * MAXN,), jnp.int32),
            pltpu.VMEM(((MAXN - 1) * NPT,), jnp.float32),
            pltpu.VMEM((NPT * MAXN,), jnp.int32),
            pltpu.VMEM((NPT * MAXN,), jnp.float32),
            pltpu.VMEM((16,), jnp.float32),
        ],
    )
    return f(aux_flat, di_flat, nb, cc)


def _sc3b_body(z_hbm, idx_hbm, cw_hbm, col_out,
               idx_v, cw_v, zb0_v, zb1_v, colstA_v, colstB_v,
               sem0, sem1, semo):
    w = _wid()
    eb = w * NPT * MAXN
    pltpu.sync_copy(idx_hbm.at[pl.ds(eb, NPT * MAXN)], idx_v)
    pltpu.sync_copy(cw_hbm.at[pl.ds(eb, NPT * MAXN)], cw_v)

    def compute(c, zbuf_v, colst_v):
        wv = [cw_v[pl.ds(c * GROWS + t * 16, 16)] for t in range(GROWS // 16)]
        for nl in range(GN):
            accs = [jnp.zeros((16,), jnp.float32) for _ in range(OUT // 16)]
            for j in range(MAXN):
                r = nl * MAXN + j
                wgt = wv[r // 16][r % 16]
                for q in range(OUT // 16):
                    accs[q] = accs[q] + zbuf_v[r, pl.ds(q * 16, 16)] * wgt
            for q in range(OUT // 16):
                colst_v[nl, pl.ds(q * 16, 16)] = accs[q]

    def fire(c, zbuf_v, sem):
        pltpu.async_copy(z_hbm.at[idx_v.at[pl.ds(c * GROWS, GROWS)]],
                         zbuf_v, sem)

    def drain(c, zbuf_v, sem):
        pltpu.make_async_copy(z_hbm.at[idx_v.at[pl.ds(c * GROWS, GROWS)]],
                              zbuf_v, sem).wait()

    def out_slice(c):
        return col_out.at[pl.ds(w * NPT + c * GN, GN)]

    fire(0, zb0_v, sem0)
    fire(1, zb1_v, sem1)

    def half(p, c, zbuf_v, gsem, colst_v):
        drain(c, zbuf_v, gsem)

        @pl.when(p > 0)
        def _():
            pltpu.make_async_copy(colst_v, out_slice(c - 2), semo).wait()

        compute(c, zbuf_v, colst_v)
        pltpu.async_copy(colst_v, out_slice(c), semo)

        @pl.when(c + 2 < NGCH)
        def _():
            fire(c + 2, zbuf_v, gsem)

    def body(p, _):
        c0 = p * 2
        half(p, c0, zb0_v, sem0, colstA_v)
        half(p, c0 + 1, zb1_v, sem1, colstB_v)
        return 0

    lax.fori_loop(0, NGCH // 2, body, 0)
    pltpu.make_async_copy(colstA_v, out_slice(NGCH - 2), semo).wait()
    pltpu.make_async_copy(colstB_v, out_slice(NGCH - 1), semo).wait()


def _sc3b(z, idx_flat, cw_flat):
    f = pl.kernel(
        _sc3b_body,
        out_type=jax.ShapeDtypeStruct((N, OUT), jnp.float32),
        mesh=_MESH,
        compiler_params=_SC_PARAMS,
        scratch_types=[
            pltpu.VMEM((NPT * MAXN,), jnp.int32),
            pltpu.VMEM((NPT * MAXN,), jnp.float32),
            pltpu.VMEM((GROWS, OUT), jnp.float32),
            pltpu.VMEM((GROWS, OUT), jnp.float32),
            pltpu.VMEM((GN, OUT), jnp.float32),
            pltpu.VMEM((GN, OUT), jnp.float32),
            pltpu.SemaphoreType.DMA,
            pltpu.SemaphoreType.DMA,
            pltpu.SemaphoreType.DMA,
        ],
    )
    return f(z, idx_flat, cw_flat)


def _tc3_body(rowt_ref, col_ref, h_ref, aggp_ref, df_ref,
              crb_ref, ccb_ref, bn1g_ref, bn1b_ref, bn2g_ref, bn2b_ref,
              gcb_ref, lw9_ref, lw128_ref, clsw_ref, clsb_ref,
              out_ref, acc_ref, st_ref):
    i = pl.program_id(0)
    nblk = pl.num_programs(0) // 2
    blk = col_ref.shape[0]

    r = rowt_ref[...] + crb_ref[0, 0]
    c = col_ref[...] + ccb_ref[0, 0]

    @pl.when(i < nblk)
    def _():
        stats = jnp.concatenate(
            [x.reshape(1, 1) for x in
             (jnp.sum(r), jnp.sum(r * r), jnp.sum(c), jnp.sum(c * c))]
            + [jnp.zeros((1, 124), jnp.float32)], axis=1)
        st_ref[...] = jnp.where(i == 0, stats, st_ref[...] + stats)

    @pl.when(i >= nblk)
    def _():
        st = st_ref[...]
        nr = float(N * (MAXN - 1))
        ncl = float(N * OUT)
        mu1 = st[0, 0] / nr
        var1 = st[0, 1] / nr - mu1 * mu1
        mu2 = st[0, 2] / ncl
        var2 = st[0, 3] / ncl - mu2 * mu2
        inv1 = bn1g_ref[0, 0] * lax.rsqrt(var1 + 1e-5)
        inv2 = bn2g_ref[0, 0] * lax.rsqrt(var2 + 1e-5)

        r1 = jnp.maximum((r - mu1) * inv1 + bn1b_ref[0, 0], 0.0)
        c1 = jnp.maximum((c - mu2) * inv2 + bn2b_ref[0, 0], 0.0)

        gat9 = lax.dot_general(r1, lw9_ref[...], (((0,), (0,)), ((), ())),
                               preferred_element_type=jnp.float32)
        gatc = jnp.dot(c1, lw128_ref[...], preferred_element_type=jnp.float32)
        feats = jnp.maximum(gat9 + gatc + h_ref[...], 0.0)

        agg = jnp.sum(aggp_ref[...], axis=0, keepdims=True)
        gc = agg * df_ref[1:2, :] + gcb_ref[0, 0]
        ng = blk // IN
        gcr = jnp.concatenate([gc[:, k * IN:(k + 1) * IN] for k in range(ng)],
                              axis=0)
        gcr = gcr - jnp.max(gcr, axis=1, keepdims=True)
        egc = jnp.exp(gcr)
        gw = egc / jnp.sum(egc, axis=1, keepdims=True)
        contrib = jnp.zeros((1, OUT), jnp.float32)
        for k in range(ng):
            contrib = contrib + jnp.dot(gw[k:k + 1, :],
                                        feats[k * IN:(k + 1) * IN, :],
                                        preferred_element_type=jnp.float32)

        newacc = jnp.where(i == nblk, contrib, acc_ref[...] + contrib)
        acc_ref[...] = newacc

        @pl.when(i == 2 * nblk - 1)
        def _():
            hg = newacc / float(N)
            out_ref[...] = lax.dot_general(
                hg, clsw_ref[...], (((1,), (1,)), ((), ())),
                preferred_element_type=jnp.float32) + clsb_ref[...]


def _tc3(rowt, col, h, aggp, dmisc_f, crb, ccb, bn1g, bn1b, bn2g,
         bn2b, gcb, lw9, lw128, clsw, clsb):
    blk = 3200
    nblk = N // blk
    grid = (2 * nblk,)
    bi = lambda i: (i % nblk, 0)
    bj = lambda i: (0, i % nblk)
    z = lambda i: (0, 0)
    return pl.pallas_call(
        _tc3_body,
        grid=grid,
        in_specs=[
            pl.BlockSpec((MAXN - 1, blk), bj),
            pl.BlockSpec((blk, OUT), bi),
            pl.BlockSpec((blk, IN), bi),
            pl.BlockSpec((TILES, blk), bj),
            pl.BlockSpec((2, blk), bj),
            pl.BlockSpec((1, 1), z),
            pl.BlockSpec((1, 1), z),
            pl.BlockSpec((1, 1), z),
            pl.BlockSpec((1, 1), z),
            pl.BlockSpec((1, 1), z),
            pl.BlockSpec((1, 1), z),
            pl.BlockSpec((1, 1), z),
            pl.BlockSpec((MAXN - 1, OUT), z),
            pl.BlockSpec((OUT, OUT), z),
            pl.BlockSpec((NC, OUT), z),
            pl.BlockSpec((1, NC), z),
        ],
        out_specs=pl.BlockSpec((1, NC), z),
        out_shape=jax.ShapeDtypeStruct((1, NC), jnp.float32),
        scratch_shapes=[pltpu.VMEM((1, OUT), jnp.float32),
                        pltpu.VMEM((1, 128), jnp.float32)],
    )(rowt, col, h, aggp, dmisc_f, crb, ccb, bn1g, bn1b, bn2g, bn2b,
      gcb, lw9, lw128, clsw, clsb)


def kernel(h, edge_index, fc_w, attn_w, convrow_w, convrow_b, bn1_g, bn1_b,
           convcol_w, convcol_b, bn2_g, bn2_b, gc_w, gc_b, localw, cls_w,
           cls_b):
    src = edge_index[0]
    dst = edge_index[1]

    a1 = attn_w[:OUT, 0]
    a2 = attn_w[OUT:, 0]
    w0 = convrow_w[0, 0, 0, :]
    w1 = convrow_w[0, 0, 1, :]
    zero = jnp.zeros((OUT,), jnp.float32)
    bmat = jnp.stack([a1, a2, w0, w1, zero, zero, zero, zero], axis=1)
    cmat = jnp.stack([zero, zero, zero, zero, gc_w[:, 0], zero, zero, zero],
                     axis=1)

    z, aux_t = _tc0(h, fc_w, bmat, cmat)

    histd_flat, hists_flat = _sc1(src, dst)
    offs, dmisc_i, dmisc_f = _tc1(histd_flat.reshape(TILES, N),
                                  hists_flat.reshape(TILES, N), aux_t)

    nb, aggp_flat = _sc2(src, dst, offs.reshape(-1), dmisc_f.reshape(-1))

    cc = jnp.pad(convcol_w[0, 0, :, 0], (0, 16 - MAXN))
    rowt_flat, idx_flat, cw_flat = _sc3a(aux_t.reshape(-1),
                                         dmisc_i.reshape(-1), nb, cc)

    col = _sc3b(z, idx_flat, cw_flat)

    rowt = rowt_flat.reshape(MAXN - 1, N)
    aggp = aggp_flat.reshape(TILES, N)

    crb = convrow_b.reshape(1, 1)
    ccb = convcol_b.reshape(1, 1)
    out = _tc3(rowt, col, h, aggp, dmisc_f, crb, ccb,
               bn1_g.reshape(1, 1), bn1_b.reshape(1, 1),
               bn2_g.reshape(1, 1), bn2_b.reshape(1, 1),
               gc_b.reshape(1, 1), localw[:MAXN - 1], localw[MAXN - 1:],
               cls_w, cls_b.reshape(1, NC))
    return out
